# baseline (speedup 1.0000x reference)
import functools

import jax
import jax.numpy as jnp
from jax import lax
from jax.experimental import pallas as pl
from jax.experimental.pallas import tpu as pltpu
from jax.experimental.pallas import tpu_sc as plsc

N = 10000
E = 320000
NC, NS, L = 2, 16, 16
NW = NC * NS
EPW = E // NW
K = 80
NCHUNK = EPW // K
ROWS_PER_SUB = N // NS
ZROUNDS = (N // K + NS - 1) // NS

_mesh = plsc.VectorSubcoreMesh(core_axis_name="c", subcore_axis_name="s")
_sc_params = pltpu.CompilerParams(use_tc_tiling_on_sc=False)


def _degrees(src3, dst3):
    out_type = (jax.ShapeDtypeStruct((NC, N, L), jnp.float32),
                jax.ShapeDtypeStruct((NC, N, L), jnp.float32))

    @functools.partial(
        pl.kernel, out_type=out_type, mesh=_mesh, compiler_params=_sc_params,
        scratch_types=[
            pltpu.VMEM((NCHUNK, K), jnp.int32),
            pltpu.VMEM((NCHUNK, K), jnp.int32),
            pltpu.VMEM((K, L), jnp.float32),
            pltpu.VMEM((K, L), jnp.float32),
            pltpu.VMEM_SHARED((N, L), jnp.float32),
            pltpu.VMEM_SHARED((N, L), jnp.float32),
            pltpu.SemaphoreType.DMA,
            pltpu.SemaphoreType.DMA,
            pltpu.SemaphoreType.DMA,
        ])
    def deg_kernel(src_hbm, dst_hbm, os_hbm, od_hbm,
                   srcv, dstv, onesv, zerov, accs, accd, sem, ssem, dsem):
        ci = lax.axis_index("c")
        si = lax.axis_index("s")
        wid = si * NC + ci

        @pl.loop(0, K)
        def _(r):
            onesv[r, :] = jnp.ones((L,), jnp.float32)
            zerov[r, :] = jnp.zeros((L,), jnp.float32)

        @pl.loop(0, ZROUNDS)
        def _(k):
            chunk = si + NS * k

            @pl.when(chunk < N // K)
            def _():
                pltpu.async_copy(zerov, accs.at[pl.ds(chunk * K, K)], sem)
                pltpu.async_copy(zerov, accd.at[pl.ds(chunk * K, K)], sem)

        @pl.loop(0, ZROUNDS)
        def _(k):
            chunk = si + NS * k

            @pl.when(chunk < N // K)
            def _():
                pltpu.make_async_copy(zerov, accs.at[pl.ds(chunk * K, K)], sem).wait()
                pltpu.make_async_copy(zerov, accd.at[pl.ds(chunk * K, K)], sem).wait()

        plsc.subcore_barrier()
        pltpu.sync_copy(src_hbm.at[wid], srcv)
        pltpu.sync_copy(dst_hbm.at[wid], dstv)

        @pl.loop(0, NCHUNK)
        def _(c):
            pltpu.async_copy(onesv, accs.at[srcv.at[c]], ssem, add=True)
            pltpu.async_copy(onesv, accd.at[dstv.at[c]], dsem, add=True)
            pltpu.make_async_copy(onesv, accs.at[srcv.at[c]], ssem).wait()
            pltpu.make_async_copy(onesv, accd.at[dstv.at[c]], dsem).wait()

        plsc.subcore_barrier()

        @pl.loop(0, ZROUNDS)
        def _(k):
            chunk = si + NS * k

            @pl.when(chunk < N // K)
            def _():
                base = chunk * K
                pltpu.async_copy(accs.at[pl.ds(base, K)],
                                 os_hbm.at[ci, pl.ds(base, K)], sem)
                pltpu.async_copy(accd.at[pl.ds(base, K)],
                                 od_hbm.at[ci, pl.ds(base, K)], sem)

        @pl.loop(0, ZROUNDS)
        def _(k):
            chunk = si + NS * k

            @pl.when(chunk < N // K)
            def _():
                base = chunk * K
                pltpu.make_async_copy(accs.at[pl.ds(base, K)],
                                      os_hbm.at[ci, pl.ds(base, K)], sem).wait()
                pltpu.make_async_copy(accd.at[pl.ds(base, K)],
                                      od_hbm.at[ci, pl.ds(base, K)], sem).wait()

    return deg_kernel(src3, dst3)


_SEC = 25
_NSEC = NCHUNK // _SEC


def _aggregate_pair(h2, src3, dst3):
    H = h2.shape[2]
    out_type = (jax.ShapeDtypeStruct((NC, N, H), jnp.float32),
                jax.ShapeDtypeStruct((NC, N, H), jnp.float32))

    @functools.partial(
        pl.kernel, out_type=out_type, mesh=_mesh, compiler_params=_sc_params,
        scratch_types=[
            pltpu.VMEM((_SEC, K), jnp.int32),
            pltpu.VMEM((_SEC, K), jnp.int32),
            pltpu.VMEM((K, H), jnp.float32),
            pltpu.VMEM((K, H), jnp.float32),
            pltpu.VMEM((K, H), jnp.float32),
            pltpu.VMEM((K, H), jnp.float32),
            pltpu.VMEM_SHARED((N, H), jnp.float32),
            pltpu.VMEM_SHARED((N, H), jnp.float32),
            pltpu.SemaphoreType.DMA,
            pltpu.SemaphoreType.DMA,
            pltpu.SemaphoreType.DMA,
            pltpu.SemaphoreType.DMA,
            pltpu.SemaphoreType.DMA,
            pltpu.SemaphoreType.DMA,
            pltpu.SemaphoreType.DMA,
            pltpu.SemaphoreType.DMA,
        ])
    def agg2_kernel(h_hbm, src_hbm, dst_hbm, olo_hbm, ohi_hbm,
                    srcv, dstv, la, lb, ha, hb, acc_lo, acc_hi,
                    sla, slb, sha, shb, ala, alb, aha, ahb):
        ci = lax.axis_index("c")
        si = lax.axis_index("s")
        wid = si * NC + ci

        @pl.loop(0, K)
        def _(r):
            @pl.loop(0, H, step=L)
            def _(j):
                la[r, pl.ds(j, L)] = jnp.zeros((L,), jnp.float32)

        @pl.loop(0, ZROUNDS)
        def _(k):
            chunk = si + NS * k

            @pl.when(chunk < N // K)
            def _():
                pltpu.async_copy(la, acc_lo.at[pl.ds(chunk * K, K)], ala)
                pltpu.async_copy(la, acc_hi.at[pl.ds(chunk * K, K)], aha)

        @pl.loop(0, ZROUNDS)
        def _(k):
            chunk = si + NS * k

            @pl.when(chunk < N // K)
            def _():
                pltpu.make_async_copy(la, acc_lo.at[pl.ds(chunk * K, K)], ala).wait()
                pltpu.make_async_copy(la, acc_hi.at[pl.ds(chunk * K, K)], aha).wait()

        plsc.subcore_barrier()

        @pl.loop(0, _NSEC)
        def _(s):
            pltpu.sync_copy(src_hbm.at[wid, pl.ds(s * _SEC, _SEC)], srcv)
            pltpu.sync_copy(dst_hbm.at[wid, pl.ds(s * _SEC, _SEC)], dstv)

            pltpu.async_copy(h_hbm.at[0].at[srcv.at[0]], la, sla)
            pltpu.async_copy(h_hbm.at[1].at[srcv.at[0]], ha, sha)
            pltpu.async_copy(h_hbm.at[0].at[srcv.at[1]], lb, slb)
            pltpu.async_copy(h_hbm.at[1].at[srcv.at[1]], hb, shb)

            @pl.loop(0, _SEC, step=2)
            def _(c):
                pltpu.make_async_copy(h_hbm.at[0].at[srcv.at[c]], la, sla).wait()
                pltpu.async_copy(la, acc_lo.at[dstv.at[c]], ala, add=True)
                pltpu.make_async_copy(h_hbm.at[1].at[srcv.at[c]], ha, sha).wait()
                pltpu.async_copy(ha, acc_hi.at[dstv.at[c]], aha, add=True)

                @pl.when(c + 1 < _SEC)
                def _():
                    pltpu.make_async_copy(h_hbm.at[0].at[srcv.at[c + 1]], lb, slb).wait()
                    pltpu.async_copy(lb, acc_lo.at[dstv.at[c + 1]], alb, add=True)
                    pltpu.make_async_copy(h_hbm.at[1].at[srcv.at[c + 1]], hb, shb).wait()
                    pltpu.async_copy(hb, acc_hi.at[dstv.at[c + 1]], ahb, add=True)

                pltpu.make_async_copy(la, acc_lo.at[dstv.at[c]], ala).wait()
                pltpu.make_async_copy(ha, acc_hi.at[dstv.at[c]], aha).wait()

                @pl.when(c + 2 < _SEC)
                def _():
                    pltpu.async_copy(h_hbm.at[0].at[srcv.at[c + 2]], la, sla)
                    pltpu.async_copy(h_hbm.at[1].at[srcv.at[c + 2]], ha, sha)

                @pl.when(c + 1 < _SEC)
                def _():
                    pltpu.make_async_copy(lb, acc_lo.at[dstv.at[c + 1]], alb).wait()
                    pltpu.make_async_copy(hb, acc_hi.at[dstv.at[c + 1]], ahb).wait()

                @pl.when(c + 3 < _SEC)
                def _():
                    pltpu.async_copy(h_hbm.at[0].at[srcv.at[c + 3]], lb, slb)
                    pltpu.async_copy(h_hbm.at[1].at[srcv.at[c + 3]], hb, shb)

        plsc.subcore_barrier()

        @pl.loop(0, ZROUNDS)
        def _(k):
            chunk = si + NS * k

            @pl.when(chunk < N // K)
            def _():
                base = chunk * K
                pltpu.async_copy(acc_lo.at[pl.ds(base, K)],
                                 olo_hbm.at[ci, pl.ds(base, K)], ala)
                pltpu.async_copy(acc_hi.at[pl.ds(base, K)],
                                 ohi_hbm.at[ci, pl.ds(base, K)], aha)

        @pl.loop(0, ZROUNDS)
        def _(k):
            chunk = si + NS * k

            @pl.when(chunk < N // K)
            def _():
                base = chunk * K
                pltpu.make_async_copy(acc_lo.at[pl.ds(base, K)],
                                      olo_hbm.at[ci, pl.ds(base, K)], ala).wait()
                pltpu.make_async_copy(acc_hi.at[pl.ds(base, K)],
                                      ohi_hbm.at[ci, pl.ds(base, K)], aha).wait()

    return agg2_kernel(h2, src3, dst3)


def _aggregate(h, src3, dst3):
    D = h.shape[1]

    @functools.partial(
        pl.kernel, out_type=jax.ShapeDtypeStruct((NC, N, D), jnp.float32),
        mesh=_mesh, compiler_params=_sc_params,
        scratch_types=[
            pltpu.VMEM((NCHUNK, K), jnp.int32),
            pltpu.VMEM((NCHUNK, K), jnp.int32),
            pltpu.VMEM((K, D), jnp.float32),
            pltpu.VMEM((K, D), jnp.float32),
            pltpu.VMEM_SHARED((N, D), jnp.float32),
            pltpu.SemaphoreType.DMA,
            pltpu.SemaphoreType.DMA,
            pltpu.SemaphoreType.DMA,
            pltpu.SemaphoreType.DMA,
        ])
    def agg_kernel(h_hbm, src_hbm, dst_hbm, out_hbm,
                   srcv, dstv, bufa, bufb, accum, sema, semb, aa, ab):
        ci = lax.axis_index("c")
        si = lax.axis_index("s")
        wid = si * NC + ci

        @pl.loop(0, K)
        def _(r):
            @pl.loop(0, D, step=L)
            def _(j):
                bufa[r, pl.ds(j, L)] = jnp.zeros((L,), jnp.float32)

        @pl.loop(0, ZROUNDS)
        def _(k):
            chunk = si + NS * k

            @pl.when(chunk < N // K)
            def _():
                pltpu.async_copy(bufa, accum.at[pl.ds(chunk * K, K)], aa)

        @pl.loop(0, ZROUNDS)
        def _(k):
            chunk = si + NS * k

            @pl.when(chunk < N // K)
            def _():
                pltpu.make_async_copy(bufa, accum.at[pl.ds(chunk * K, K)], aa).wait()

        plsc.subcore_barrier()
        pltpu.sync_copy(src_hbm.at[wid], srcv)
        pltpu.sync_copy(dst_hbm.at[wid], dstv)

        pltpu.async_copy(h_hbm.at[srcv.at[0]], bufa, sema)
        pltpu.async_copy(h_hbm.at[srcv.at[1]], bufb, semb)

        @pl.loop(0, NCHUNK, step=2)
        def _(c):
            pltpu.make_async_copy(h_hbm.at[srcv.at[c]], bufa, sema).wait()
            pltpu.async_copy(bufa, accum.at[dstv.at[c]], aa, add=True)

            @pl.when(c + 1 < NCHUNK)
            def _():
                pltpu.make_async_copy(h_hbm.at[srcv.at[c + 1]], bufb, semb).wait()
                pltpu.async_copy(bufb, accum.at[dstv.at[c + 1]], ab, add=True)

            pltpu.make_async_copy(bufa, accum.at[dstv.at[c]], aa).wait()

            @pl.when(c + 2 < NCHUNK)
            def _():
                pltpu.async_copy(h_hbm.at[srcv.at[c + 2]], bufa, sema)

            @pl.when(c + 1 < NCHUNK)
            def _():
                pltpu.make_async_copy(bufb, accum.at[dstv.at[c + 1]], ab).wait()

            @pl.when(c + 3 < NCHUNK)
            def _():
                pltpu.async_copy(h_hbm.at[srcv.at[c + 3]], bufb, semb)

        plsc.subcore_barrier()

        @pl.loop(0, ZROUNDS)
        def _(k):
            chunk = si + NS * k

            @pl.when(chunk < N // K)
            def _():
                base = chunk * K
                pltpu.async_copy(accum.at[pl.ds(base, K)],
                                 out_hbm.at[ci, pl.ds(base, K)], aa)

        @pl.loop(0, ZROUNDS)
        def _(k):
            chunk = si + NS * k

            @pl.when(chunk < N // K)
            def _():
                base = chunk * K
                pltpu.make_async_copy(accum.at[pl.ds(base, K)],
                                      out_hbm.at[ci, pl.ds(base, K)], aa).wait()

    return agg_kernel(h, src3, dst3)


_R = 1000


def _norm_from(counts_ref):
    c = counts_ref[0, :, 0:1] + counts_ref[1, :, 0:1]
    return lax.rsqrt(jnp.maximum(c, 1.0))


def _scale_matmul(x, cs, W):
    D, Do = W.shape
    H = Do // 2

    def body(x_ref, cs_ref, w_ref, o_ref):
        d = jnp.dot(x_ref[...] * _norm_from(cs_ref), w_ref[...],
                    preferred_element_type=jnp.float32)
        o_ref[0] = d[:, :H]
        o_ref[1] = d[:, H:]

    return pl.pallas_call(
        body, grid=(N // _R,),
        in_specs=[pl.BlockSpec((_R, D), lambda i: (i, 0)),
                  pl.BlockSpec((NC, _R, L), lambda i: (0, i, 0)),
                  pl.BlockSpec((D, Do), lambda i: (0, 0))],
        out_specs=pl.BlockSpec((2, _R, H), lambda i: (0, i, 0)),
        out_shape=jax.ShapeDtypeStruct((2, N, H), jnp.float32))(x, cs, W)


def _update_matmul(a_lo, a_hi, cd, cs, b, W, split_out):
    D, Do = W.shape
    H = D // 2
    Ho = Do // 2

    def body(alo_ref, ahi_ref, cd_ref, cs_ref, b_ref, w_ref, o_ref):
        ndst = _norm_from(cd_ref)
        nsrc = _norm_from(cs_ref)
        h_lo = jnp.maximum((alo_ref[0] + alo_ref[1]) * ndst
                           + b_ref[:, :H], 0.0) * nsrc
        h_hi = jnp.maximum((ahi_ref[0] + ahi_ref[1]) * ndst
                           + b_ref[:, H:], 0.0) * nsrc
        d = (jnp.dot(h_lo, w_ref[:H], preferred_element_type=jnp.float32)
             + jnp.dot(h_hi, w_ref[H:], preferred_element_type=jnp.float32))
        if split_out:
            o_ref[0] = d[:, :Ho]
            o_ref[1] = d[:, Ho:]
        else:
            o_ref[...] = d

    if split_out:
        out_spec = pl.BlockSpec((2, _R, Ho), lambda i: (0, i, 0))
        out_shape = jax.ShapeDtypeStruct((2, N, Ho), jnp.float32)
    else:
        out_spec = pl.BlockSpec((_R, Do), lambda i: (i, 0))
        out_shape = jax.ShapeDtypeStruct((N, Do), jnp.float32)

    return pl.pallas_call(
        body, grid=(N // _R,),
        in_specs=[pl.BlockSpec((NC, _R, H), lambda i: (0, i, 0)),
                  pl.BlockSpec((NC, _R, H), lambda i: (0, i, 0)),
                  pl.BlockSpec((NC, _R, L), lambda i: (0, i, 0)),
                  pl.BlockSpec((NC, _R, L), lambda i: (0, i, 0)),
                  pl.BlockSpec((1, D), lambda i: (0, 0)),
                  pl.BlockSpec((D, Do), lambda i: (0, 0))],
        out_specs=out_spec,
        out_shape=out_shape)(a_lo, a_hi, cd, cs, b, W)


def _finalize(agg, cd, b):
    D = agg.shape[2]

    def body(a_ref, cd_ref, b_ref, o_ref):
        a = a_ref[0] + a_ref[1]
        o_ref[...] = a * _norm_from(cd_ref) + b_ref[...]

    return pl.pallas_call(
        body, grid=(N // _R,),
        in_specs=[pl.BlockSpec((NC, _R, D), lambda i: (0, i, 0)),
                  pl.BlockSpec((NC, _R, L), lambda i: (0, i, 0)),
                  pl.BlockSpec((1, D), lambda i: (0, 0))],
        out_specs=pl.BlockSpec((_R, D), lambda i: (i, 0)),
        out_shape=jax.ShapeDtypeStruct((N, D), jnp.float32))(agg, cd, b)


def kernel(features, edge_index, W1, b1, W2, b2, W3, b3):
    src3 = edge_index[0].astype(jnp.int32).reshape(NW, NCHUNK, K)
    dst3 = edge_index[1].astype(jnp.int32).reshape(NW, NCHUNK, K)

    cs, cd = _degrees(src3, dst3)

    h0 = _scale_matmul(features, cs, W1)
    a1lo, a1hi = _aggregate_pair(h0, src3, dst3)
    h1 = _update_matmul(a1lo, a1hi, cd, cs, b1.reshape(1, -1), W2,
                        split_out=True)
    a2lo, a2hi = _aggregate_pair(h1, src3, dst3)

    W3p = jnp.pad(W3, ((0, 0), (0, 8)))
    b3p = jnp.pad(b3, (0, 8))
    h2 = _update_matmul(a2lo, a2hi, cd, cs, b2.reshape(1, -1), W3p,
                        split_out=False)
    a3 = _aggregate(h2, src3, dst3)
    out = _finalize(a3, cd, b3p.reshape(1, -1))
    return out[:, :40]


# speedup vs baseline: 1.0619x; 1.0619x over previous
import functools

import jax
import jax.numpy as jnp
from jax import lax
from jax.experimental import pallas as pl
from jax.experimental.pallas import tpu as pltpu
from jax.experimental.pallas import tpu_sc as plsc

N = 10000
E = 320000
NC, NS, L = 2, 16, 16
NW = NC * NS
EPW = E // NW
K = 80
NCHUNK = EPW // K
ROWS_PER_SUB = N // NS
ZROUNDS = (N // K + NS - 1) // NS

_mesh = plsc.VectorSubcoreMesh(core_axis_name="c", subcore_axis_name="s")
_sc_params = pltpu.CompilerParams(use_tc_tiling_on_sc=False)


def _degrees(src3, dst3):
    out_type = (jax.ShapeDtypeStruct((NC, N, L), jnp.float32),
                jax.ShapeDtypeStruct((NC, N, L), jnp.float32))

    @functools.partial(
        pl.kernel, out_type=out_type, mesh=_mesh, compiler_params=_sc_params,
        scratch_types=[
            pltpu.VMEM((NCHUNK, K), jnp.int32),
            pltpu.VMEM((NCHUNK, K), jnp.int32),
            pltpu.VMEM((K, L), jnp.float32),
            pltpu.VMEM((K, L), jnp.float32),
            pltpu.VMEM_SHARED((N, L), jnp.float32),
            pltpu.VMEM_SHARED((N, L), jnp.float32),
            pltpu.SemaphoreType.DMA,
            pltpu.SemaphoreType.DMA,
            pltpu.SemaphoreType.DMA,
        ])
    def deg_kernel(src_hbm, dst_hbm, os_hbm, od_hbm,
                   srcv, dstv, onesv, zerov, accs, accd, sem, ssem, dsem):
        ci = lax.axis_index("c")
        si = lax.axis_index("s")
        wid = si * NC + ci

        @pl.loop(0, K)
        def _(r):
            onesv[r, :] = jnp.ones((L,), jnp.float32)
            zerov[r, :] = jnp.zeros((L,), jnp.float32)

        @pl.loop(0, ZROUNDS)
        def _(k):
            chunk = si + NS * k

            @pl.when(chunk < N // K)
            def _():
                pltpu.async_copy(zerov, accs.at[pl.ds(chunk * K, K)], sem)
                pltpu.async_copy(zerov, accd.at[pl.ds(chunk * K, K)], sem)

        @pl.loop(0, ZROUNDS)
        def _(k):
            chunk = si + NS * k

            @pl.when(chunk < N // K)
            def _():
                pltpu.make_async_copy(zerov, accs.at[pl.ds(chunk * K, K)], sem).wait()
                pltpu.make_async_copy(zerov, accd.at[pl.ds(chunk * K, K)], sem).wait()

        plsc.subcore_barrier()
        pltpu.sync_copy(src_hbm.at[wid], srcv)
        pltpu.sync_copy(dst_hbm.at[wid], dstv)

        @pl.loop(0, NCHUNK)
        def _(c):
            pltpu.async_copy(onesv, accs.at[srcv.at[c]], ssem, add=True)
            pltpu.async_copy(onesv, accd.at[dstv.at[c]], dsem, add=True)
            pltpu.make_async_copy(onesv, accs.at[srcv.at[c]], ssem).wait()
            pltpu.make_async_copy(onesv, accd.at[dstv.at[c]], dsem).wait()

        plsc.subcore_barrier()

        @pl.loop(0, ZROUNDS)
        def _(k):
            chunk = si + NS * k

            @pl.when(chunk < N // K)
            def _():
                base = chunk * K
                pltpu.async_copy(accs.at[pl.ds(base, K)],
                                 os_hbm.at[ci, pl.ds(base, K)], sem)
                pltpu.async_copy(accd.at[pl.ds(base, K)],
                                 od_hbm.at[ci, pl.ds(base, K)], sem)

        @pl.loop(0, ZROUNDS)
        def _(k):
            chunk = si + NS * k

            @pl.when(chunk < N // K)
            def _():
                base = chunk * K
                pltpu.make_async_copy(accs.at[pl.ds(base, K)],
                                      os_hbm.at[ci, pl.ds(base, K)], sem).wait()
                pltpu.make_async_copy(accd.at[pl.ds(base, K)],
                                      od_hbm.at[ci, pl.ds(base, K)], sem).wait()

    return deg_kernel(src3, dst3)


_SEC = 25
_NSEC = NCHUNK // _SEC
_D = 128


def _aggregate_full(h2, src3, dst3):
    H = _D // 2
    out_type = jax.ShapeDtypeStruct((NC, N, _D), jnp.float32)

    @functools.partial(
        pl.kernel, out_type=out_type, mesh=_mesh, compiler_params=_sc_params,
        scratch_types=[
            pltpu.VMEM((_SEC, K), jnp.int32),
            pltpu.VMEM((_SEC, K), jnp.int32),
            pltpu.VMEM((K, H), jnp.float32),
            pltpu.VMEM((K, H), jnp.float32),
            pltpu.VMEM((K, H), jnp.float32),
            pltpu.VMEM((K, H), jnp.float32),
            pltpu.VMEM_SHARED((N, H), jnp.float32),
            pltpu.VMEM_SHARED((N, H), jnp.float32),
            pltpu.SemaphoreType.DMA,
            pltpu.SemaphoreType.DMA,
            pltpu.SemaphoreType.DMA,
            pltpu.SemaphoreType.DMA,
            pltpu.SemaphoreType.DMA,
            pltpu.SemaphoreType.DMA,
            pltpu.SemaphoreType.DMA,
            pltpu.SemaphoreType.DMA,
        ])
    def agg2_kernel(h_hbm, src_hbm, dst_hbm, out_hbm,
                    srcv, dstv, la, lb, ha, hb, acc_lo, acc_hi,
                    sla, slb, sha, shb, ala, alb, aha, ahb):
        ci = lax.axis_index("c")
        si = lax.axis_index("s")
        wid = si * NC + ci

        @pl.loop(0, K)
        def _(r):
            @pl.loop(0, H, step=L)
            def _(j):
                la[r, pl.ds(j, L)] = jnp.zeros((L,), jnp.float32)

        @pl.loop(0, ZROUNDS)
        def _(k):
            chunk = si + NS * k

            @pl.when(chunk < N // K)
            def _():
                pltpu.async_copy(la, acc_lo.at[pl.ds(chunk * K, K)], ala)
                pltpu.async_copy(la, acc_hi.at[pl.ds(chunk * K, K)], aha)

        @pl.loop(0, ZROUNDS)
        def _(k):
            chunk = si + NS * k

            @pl.when(chunk < N // K)
            def _():
                pltpu.make_async_copy(la, acc_lo.at[pl.ds(chunk * K, K)], ala).wait()
                pltpu.make_async_copy(la, acc_hi.at[pl.ds(chunk * K, K)], aha).wait()

        plsc.subcore_barrier()

        @pl.loop(0, _NSEC)
        def _(s):
            pltpu.sync_copy(src_hbm.at[wid, pl.ds(s * _SEC, _SEC)], srcv)
            pltpu.sync_copy(dst_hbm.at[wid, pl.ds(s * _SEC, _SEC)], dstv)

            pltpu.async_copy(h_hbm.at[0].at[srcv.at[0]], la, sla)
            pltpu.async_copy(h_hbm.at[1].at[srcv.at[0]], ha, sha)
            pltpu.async_copy(h_hbm.at[0].at[srcv.at[1]], lb, slb)
            pltpu.async_copy(h_hbm.at[1].at[srcv.at[1]], hb, shb)

            @pl.loop(0, _SEC, step=2)
            def _(c):
                pltpu.make_async_copy(h_hbm.at[0].at[srcv.at[c]], la, sla).wait()
                pltpu.async_copy(la, acc_lo.at[dstv.at[c]], ala, add=True)
                pltpu.make_async_copy(h_hbm.at[1].at[srcv.at[c]], ha, sha).wait()
                pltpu.async_copy(ha, acc_hi.at[dstv.at[c]], aha, add=True)

                @pl.when(c + 1 < _SEC)
                def _():
                    pltpu.make_async_copy(h_hbm.at[0].at[srcv.at[c + 1]], lb, slb).wait()
                    pltpu.async_copy(lb, acc_lo.at[dstv.at[c + 1]], alb, add=True)
                    pltpu.make_async_copy(h_hbm.at[1].at[srcv.at[c + 1]], hb, shb).wait()
                    pltpu.async_copy(hb, acc_hi.at[dstv.at[c + 1]], ahb, add=True)

                pltpu.make_async_copy(la, acc_lo.at[dstv.at[c]], ala).wait()
                pltpu.make_async_copy(ha, acc_hi.at[dstv.at[c]], aha).wait()

                @pl.when(c + 2 < _SEC)
                def _():
                    pltpu.async_copy(h_hbm.at[0].at[srcv.at[c + 2]], la, sla)
                    pltpu.async_copy(h_hbm.at[1].at[srcv.at[c + 2]], ha, sha)

                @pl.when(c + 1 < _SEC)
                def _():
                    pltpu.make_async_copy(lb, acc_lo.at[dstv.at[c + 1]], alb).wait()
                    pltpu.make_async_copy(hb, acc_hi.at[dstv.at[c + 1]], ahb).wait()

                @pl.when(c + 3 < _SEC)
                def _():
                    pltpu.async_copy(h_hbm.at[0].at[srcv.at[c + 3]], lb, slb)
                    pltpu.async_copy(h_hbm.at[1].at[srcv.at[c + 3]], hb, shb)

        plsc.subcore_barrier()

        @pl.loop(0, ZROUNDS)
        def _(k):
            chunk = si + NS * k

            @pl.when(chunk < N // K)
            def _():
                base = chunk * K
                pltpu.async_copy(acc_lo.at[pl.ds(base, K)],
                                 out_hbm.at[ci, pl.ds(base, K), pl.ds(0, H)], ala)
                pltpu.async_copy(acc_hi.at[pl.ds(base, K)],
                                 out_hbm.at[ci, pl.ds(base, K), pl.ds(H, H)], aha)

        @pl.loop(0, ZROUNDS)
        def _(k):
            chunk = si + NS * k

            @pl.when(chunk < N // K)
            def _():
                base = chunk * K
                pltpu.make_async_copy(
                    acc_lo.at[pl.ds(base, K)],
                    out_hbm.at[ci, pl.ds(base, K), pl.ds(0, H)], ala).wait()
                pltpu.make_async_copy(
                    acc_hi.at[pl.ds(base, K)],
                    out_hbm.at[ci, pl.ds(base, K), pl.ds(H, H)], aha).wait()

    return agg2_kernel(h2, src3, dst3)


def _aggregate(h, src3, dst3):
    D = h.shape[1]

    @functools.partial(
        pl.kernel, out_type=jax.ShapeDtypeStruct((NC, N, D), jnp.float32),
        mesh=_mesh, compiler_params=_sc_params,
        scratch_types=[
            pltpu.VMEM((NCHUNK, K), jnp.int32),
            pltpu.VMEM((NCHUNK, K), jnp.int32),
            pltpu.VMEM((K, D), jnp.float32),
            pltpu.VMEM((K, D), jnp.float32),
            pltpu.VMEM_SHARED((N, D), jnp.float32),
            pltpu.SemaphoreType.DMA,
            pltpu.SemaphoreType.DMA,
            pltpu.SemaphoreType.DMA,
            pltpu.SemaphoreType.DMA,
        ])
    def agg_kernel(h_hbm, src_hbm, dst_hbm, out_hbm,
                   srcv, dstv, bufa, bufb, accum, sema, semb, aa, ab):
        ci = lax.axis_index("c")
        si = lax.axis_index("s")
        wid = si * NC + ci

        @pl.loop(0, K)
        def _(r):
            @pl.loop(0, D, step=L)
            def _(j):
                bufa[r, pl.ds(j, L)] = jnp.zeros((L,), jnp.float32)

        @pl.loop(0, ZROUNDS)
        def _(k):
            chunk = si + NS * k

            @pl.when(chunk < N // K)
            def _():
                pltpu.async_copy(bufa, accum.at[pl.ds(chunk * K, K)], aa)

        @pl.loop(0, ZROUNDS)
        def _(k):
            chunk = si + NS * k

            @pl.when(chunk < N // K)
            def _():
                pltpu.make_async_copy(bufa, accum.at[pl.ds(chunk * K, K)], aa).wait()

        plsc.subcore_barrier()
        pltpu.sync_copy(src_hbm.at[wid], srcv)
        pltpu.sync_copy(dst_hbm.at[wid], dstv)

        pltpu.async_copy(h_hbm.at[srcv.at[0]], bufa, sema)
        pltpu.async_copy(h_hbm.at[srcv.at[1]], bufb, semb)

        @pl.loop(0, NCHUNK, step=2)
        def _(c):
            pltpu.make_async_copy(h_hbm.at[srcv.at[c]], bufa, sema).wait()
            pltpu.async_copy(bufa, accum.at[dstv.at[c]], aa, add=True)

            @pl.when(c + 1 < NCHUNK)
            def _():
                pltpu.make_async_copy(h_hbm.at[srcv.at[c + 1]], bufb, semb).wait()
                pltpu.async_copy(bufb, accum.at[dstv.at[c + 1]], ab, add=True)

            pltpu.make_async_copy(bufa, accum.at[dstv.at[c]], aa).wait()

            @pl.when(c + 2 < NCHUNK)
            def _():
                pltpu.async_copy(h_hbm.at[srcv.at[c + 2]], bufa, sema)

            @pl.when(c + 1 < NCHUNK)
            def _():
                pltpu.make_async_copy(bufb, accum.at[dstv.at[c + 1]], ab).wait()

            @pl.when(c + 3 < NCHUNK)
            def _():
                pltpu.async_copy(h_hbm.at[srcv.at[c + 3]], bufb, semb)

        plsc.subcore_barrier()

        @pl.loop(0, ZROUNDS)
        def _(k):
            chunk = si + NS * k

            @pl.when(chunk < N // K)
            def _():
                base = chunk * K
                pltpu.async_copy(accum.at[pl.ds(base, K)],
                                 out_hbm.at[ci, pl.ds(base, K)], aa)

        @pl.loop(0, ZROUNDS)
        def _(k):
            chunk = si + NS * k

            @pl.when(chunk < N // K)
            def _():
                base = chunk * K
                pltpu.make_async_copy(accum.at[pl.ds(base, K)],
                                      out_hbm.at[ci, pl.ds(base, K)], aa).wait()

    return agg_kernel(h, src3, dst3)


_R = 1000


def _norm_from(counts_ref):
    c = counts_ref[0, :, 0:1] + counts_ref[1, :, 0:1]
    return lax.rsqrt(jnp.maximum(c, 1.0))


def _scale_matmul(x, cs, W):
    D, Do = W.shape
    H = Do // 2

    def body(x_ref, cs_ref, w_ref, o_ref):
        d = jnp.dot(x_ref[...] * _norm_from(cs_ref), w_ref[...],
                    preferred_element_type=jnp.float32)
        o_ref[0] = d[:, :H]
        o_ref[1] = d[:, H:]

    return pl.pallas_call(
        body, grid=(N // _R,),
        in_specs=[pl.BlockSpec((_R, D), lambda i: (i, 0)),
                  pl.BlockSpec((NC, _R, L), lambda i: (0, i, 0)),
                  pl.BlockSpec((D, Do), lambda i: (0, 0))],
        out_specs=pl.BlockSpec((2, _R, H), lambda i: (0, i, 0)),
        out_shape=jax.ShapeDtypeStruct((2, N, H), jnp.float32))(x, cs, W)


def _update_matmul(agg, cd, cs, b, W, split_out):
    D, Do = W.shape
    Ho = Do // 2

    def body(a_ref, cd_ref, cs_ref, b_ref, w_ref, o_ref):
        a = a_ref[0] + a_ref[1]
        h = jnp.maximum(a * _norm_from(cd_ref) + b_ref[...], 0.0) \
            * _norm_from(cs_ref)
        d = jnp.dot(h, w_ref[...], preferred_element_type=jnp.float32)
        if split_out:
            o_ref[0] = d[:, :Ho]
            o_ref[1] = d[:, Ho:]
        else:
            o_ref[...] = d

    if split_out:
        out_spec = pl.BlockSpec((2, _R, Ho), lambda i: (0, i, 0))
        out_shape = jax.ShapeDtypeStruct((2, N, Ho), jnp.float32)
    else:
        out_spec = pl.BlockSpec((_R, Do), lambda i: (i, 0))
        out_shape = jax.ShapeDtypeStruct((N, Do), jnp.float32)

    return pl.pallas_call(
        body, grid=(N // _R,),
        in_specs=[pl.BlockSpec((NC, _R, D), lambda i: (0, i, 0)),
                  pl.BlockSpec((NC, _R, L), lambda i: (0, i, 0)),
                  pl.BlockSpec((NC, _R, L), lambda i: (0, i, 0)),
                  pl.BlockSpec((1, D), lambda i: (0, 0)),
                  pl.BlockSpec((D, Do), lambda i: (0, 0))],
        out_specs=out_spec,
        out_shape=out_shape)(agg, cd, cs, b, W)


def _finalize(agg, cd, b):
    D = agg.shape[2]
    Do = 40

    def body(a_ref, cd_ref, b_ref, o_ref):
        a = a_ref[0] + a_ref[1]
        o_ref[...] = (a * _norm_from(cd_ref) + b_ref[...])[:, :Do]

    return pl.pallas_call(
        body, grid=(N // _R,),
        in_specs=[pl.BlockSpec((NC, _R, D), lambda i: (0, i, 0)),
                  pl.BlockSpec((NC, _R, L), lambda i: (0, i, 0)),
                  pl.BlockSpec((1, D), lambda i: (0, 0))],
        out_specs=pl.BlockSpec((_R, Do), lambda i: (i, 0)),
        out_shape=jax.ShapeDtypeStruct((N, Do), jnp.float32))(agg, cd, b)


def kernel(features, edge_index, W1, b1, W2, b2, W3, b3):
    src3 = edge_index[0].astype(jnp.int32).reshape(NW, NCHUNK, K)
    dst3 = edge_index[1].astype(jnp.int32).reshape(NW, NCHUNK, K)

    cs, cd = _degrees(src3, dst3)

    h0 = _scale_matmul(features, cs, W1)
    a1 = _aggregate_full(h0, src3, dst3)
    h1 = _update_matmul(a1, cd, cs, b1.reshape(1, -1), W2,
                        split_out=True)
    a2 = _aggregate_full(h1, src3, dst3)

    W3p = jnp.pad(W3, ((0, 0), (0, 8)))
    b3p = jnp.pad(b3, (0, 8))
    h2 = _update_matmul(a2, cd, cs, b2.reshape(1, -1), W3p,
                        split_out=False)
    a3 = _aggregate(h2, src3, dst3)
    out = _finalize(a3, cd, b3p.reshape(1, -1))
    return out


# speedup vs baseline: 1.0789x; 1.0160x over previous
import functools

import jax
import jax.numpy as jnp
from jax import lax
from jax.experimental import pallas as pl
from jax.experimental.pallas import tpu as pltpu
from jax.experimental.pallas import tpu_sc as plsc

N = 10000
E = 320000
NC, NS, L = 2, 16, 16
NW = NC * NS
EPW = E // NW
K = 80
NCHUNK = EPW // K
ROWS_PER_SUB = N // NS
ZROUNDS = (N // K + NS - 1) // NS

_mesh = plsc.VectorSubcoreMesh(core_axis_name="c", subcore_axis_name="s")
_sc_params = pltpu.CompilerParams(use_tc_tiling_on_sc=False)


def _degrees(src3, dst3):
    out_type = (jax.ShapeDtypeStruct((NC, N, L), jnp.float32),
                jax.ShapeDtypeStruct((NC, N, L), jnp.float32))

    @functools.partial(
        pl.kernel, out_type=out_type, mesh=_mesh, compiler_params=_sc_params,
        scratch_types=[
            pltpu.VMEM((NCHUNK, K), jnp.int32),
            pltpu.VMEM((NCHUNK, K), jnp.int32),
            pltpu.VMEM((K, L), jnp.float32),
            pltpu.VMEM((K, L), jnp.float32),
            pltpu.VMEM_SHARED((N, L), jnp.float32),
            pltpu.VMEM_SHARED((N, L), jnp.float32),
            pltpu.SemaphoreType.DMA,
            pltpu.SemaphoreType.DMA,
            pltpu.SemaphoreType.DMA,
        ])
    def deg_kernel(src_hbm, dst_hbm, os_hbm, od_hbm,
                   srcv, dstv, onesv, zerov, accs, accd, sem, ssem, dsem):
        ci = lax.axis_index("c")
        si = lax.axis_index("s")
        wid = si * NC + ci

        @pl.loop(0, K)
        def _(r):
            onesv[r, :] = jnp.ones((L,), jnp.float32)
            zerov[r, :] = jnp.zeros((L,), jnp.float32)

        @pl.loop(0, ZROUNDS)
        def _(k):
            chunk = si + NS * k

            @pl.when(chunk < N // K)
            def _():
                pltpu.async_copy(zerov, accs.at[pl.ds(chunk * K, K)], sem)
                pltpu.async_copy(zerov, accd.at[pl.ds(chunk * K, K)], sem)

        @pl.loop(0, ZROUNDS)
        def _(k):
            chunk = si + NS * k

            @pl.when(chunk < N // K)
            def _():
                pltpu.make_async_copy(zerov, accs.at[pl.ds(chunk * K, K)], sem).wait()
                pltpu.make_async_copy(zerov, accd.at[pl.ds(chunk * K, K)], sem).wait()

        plsc.subcore_barrier()
        pltpu.sync_copy(src_hbm.at[wid], srcv)
        pltpu.sync_copy(dst_hbm.at[wid], dstv)

        @pl.loop(0, NCHUNK)
        def _(c):
            pltpu.async_copy(onesv, accs.at[srcv.at[c]], ssem, add=True)
            pltpu.async_copy(onesv, accd.at[dstv.at[c]], dsem, add=True)
            pltpu.make_async_copy(onesv, accs.at[srcv.at[c]], ssem).wait()
            pltpu.make_async_copy(onesv, accd.at[dstv.at[c]], dsem).wait()

        plsc.subcore_barrier()

        @pl.loop(0, ZROUNDS)
        def _(k):
            chunk = si + NS * k

            @pl.when(chunk < N // K)
            def _():
                base = chunk * K
                pltpu.async_copy(accs.at[pl.ds(base, K)],
                                 os_hbm.at[ci, pl.ds(base, K)], sem)
                pltpu.async_copy(accd.at[pl.ds(base, K)],
                                 od_hbm.at[ci, pl.ds(base, K)], sem)

        @pl.loop(0, ZROUNDS)
        def _(k):
            chunk = si + NS * k

            @pl.when(chunk < N // K)
            def _():
                base = chunk * K
                pltpu.make_async_copy(accs.at[pl.ds(base, K)],
                                      os_hbm.at[ci, pl.ds(base, K)], sem).wait()
                pltpu.make_async_copy(accd.at[pl.ds(base, K)],
                                      od_hbm.at[ci, pl.ds(base, K)], sem).wait()

    return deg_kernel(src3, dst3)


_SEC = 25
_NSEC = NCHUNK // _SEC
_D = 128


def _aggregate_full(h2, srclo3, srchi3, dst3):
    H = _D // 2
    out_type = jax.ShapeDtypeStruct((NC, N, _D), jnp.float32)

    @functools.partial(
        pl.kernel, out_type=out_type, mesh=_mesh, compiler_params=_sc_params,
        scratch_types=[
            pltpu.VMEM((_SEC, K), jnp.int32),
            pltpu.VMEM((_SEC, K), jnp.int32),
            pltpu.VMEM((_SEC, K), jnp.int32),
            pltpu.VMEM((K, H), jnp.float32),
            pltpu.VMEM((K, H), jnp.float32),
            pltpu.VMEM((K, H), jnp.float32),
            pltpu.VMEM((K, H), jnp.float32),
            pltpu.VMEM_SHARED((N, H), jnp.float32),
            pltpu.VMEM_SHARED((N, H), jnp.float32),
            pltpu.SemaphoreType.DMA,
            pltpu.SemaphoreType.DMA,
            pltpu.SemaphoreType.DMA,
            pltpu.SemaphoreType.DMA,
            pltpu.SemaphoreType.DMA,
            pltpu.SemaphoreType.DMA,
            pltpu.SemaphoreType.DMA,
            pltpu.SemaphoreType.DMA,
        ])
    def agg2_kernel(h_hbm, srclo_hbm, srchi_hbm, dst_hbm, out_hbm,
                    srclov, srchiv, dstv, la, lb, ha, hb, acc_lo, acc_hi,
                    sla, slb, sha, shb, ala, alb, aha, ahb):
        ci = lax.axis_index("c")
        si = lax.axis_index("s")
        wid = si * NC + ci

        @pl.loop(0, K)
        def _(r):
            @pl.loop(0, H, step=L)
            def _(j):
                la[r, pl.ds(j, L)] = jnp.zeros((L,), jnp.float32)

        @pl.loop(0, ZROUNDS)
        def _(k):
            chunk = si + NS * k

            @pl.when(chunk < N // K)
            def _():
                pltpu.async_copy(la, acc_lo.at[pl.ds(chunk * K, K)], ala)
                pltpu.async_copy(la, acc_hi.at[pl.ds(chunk * K, K)], aha)

        @pl.loop(0, ZROUNDS)
        def _(k):
            chunk = si + NS * k

            @pl.when(chunk < N // K)
            def _():
                pltpu.make_async_copy(la, acc_lo.at[pl.ds(chunk * K, K)], ala).wait()
                pltpu.make_async_copy(la, acc_hi.at[pl.ds(chunk * K, K)], aha).wait()

        plsc.subcore_barrier()

        @pl.loop(0, _NSEC)
        def _(s):
            pltpu.sync_copy(srclo_hbm.at[wid, pl.ds(s * _SEC, _SEC)], srclov)
            pltpu.sync_copy(srchi_hbm.at[wid, pl.ds(s * _SEC, _SEC)], srchiv)
            pltpu.sync_copy(dst_hbm.at[wid, pl.ds(s * _SEC, _SEC)], dstv)

            pltpu.async_copy(h_hbm.at[srclov.at[0]], la, sla)
            pltpu.async_copy(h_hbm.at[srchiv.at[0]], ha, sha)
            pltpu.async_copy(h_hbm.at[srclov.at[1]], lb, slb)
            pltpu.async_copy(h_hbm.at[srchiv.at[1]], hb, shb)

            @pl.loop(0, _SEC, step=2)
            def _(c):
                pltpu.make_async_copy(h_hbm.at[srclov.at[c]], la, sla).wait()
                pltpu.async_copy(la, acc_lo.at[dstv.at[c]], ala, add=True)
                pltpu.make_async_copy(h_hbm.at[srchiv.at[c]], ha, sha).wait()
                pltpu.async_copy(ha, acc_hi.at[dstv.at[c]], aha, add=True)

                @pl.when(c + 1 < _SEC)
                def _():
                    pltpu.make_async_copy(h_hbm.at[srclov.at[c + 1]], lb, slb).wait()
                    pltpu.async_copy(lb, acc_lo.at[dstv.at[c + 1]], alb, add=True)
                    pltpu.make_async_copy(h_hbm.at[srchiv.at[c + 1]], hb, shb).wait()
                    pltpu.async_copy(hb, acc_hi.at[dstv.at[c + 1]], ahb, add=True)

                pltpu.make_async_copy(la, acc_lo.at[dstv.at[c]], ala).wait()
                pltpu.make_async_copy(ha, acc_hi.at[dstv.at[c]], aha).wait()

                @pl.when(c + 2 < _SEC)
                def _():
                    pltpu.async_copy(h_hbm.at[srclov.at[c + 2]], la, sla)
                    pltpu.async_copy(h_hbm.at[srchiv.at[c + 2]], ha, sha)

                @pl.when(c + 1 < _SEC)
                def _():
                    pltpu.make_async_copy(lb, acc_lo.at[dstv.at[c + 1]], alb).wait()
                    pltpu.make_async_copy(hb, acc_hi.at[dstv.at[c + 1]], ahb).wait()

                @pl.when(c + 3 < _SEC)
                def _():
                    pltpu.async_copy(h_hbm.at[srclov.at[c + 3]], lb, slb)
                    pltpu.async_copy(h_hbm.at[srchiv.at[c + 3]], hb, shb)

        plsc.subcore_barrier()

        @pl.loop(0, ZROUNDS)
        def _(k):
            chunk = si + NS * k

            @pl.when(chunk < N // K)
            def _():
                base = chunk * K
                pltpu.async_copy(acc_lo.at[pl.ds(base, K)],
                                 out_hbm.at[ci, pl.ds(base, K), pl.ds(0, H)], ala)
                pltpu.async_copy(acc_hi.at[pl.ds(base, K)],
                                 out_hbm.at[ci, pl.ds(base, K), pl.ds(H, H)], aha)

        @pl.loop(0, ZROUNDS)
        def _(k):
            chunk = si + NS * k

            @pl.when(chunk < N // K)
            def _():
                base = chunk * K
                pltpu.make_async_copy(
                    acc_lo.at[pl.ds(base, K)],
                    out_hbm.at[ci, pl.ds(base, K), pl.ds(0, H)], ala).wait()
                pltpu.make_async_copy(
                    acc_hi.at[pl.ds(base, K)],
                    out_hbm.at[ci, pl.ds(base, K), pl.ds(H, H)], aha).wait()

    return agg2_kernel(h2, srclo3, srchi3, dst3)


def _aggregate(h, src3, dst3):
    D = h.shape[1]

    @functools.partial(
        pl.kernel, out_type=jax.ShapeDtypeStruct((NC, N, D), jnp.float32),
        mesh=_mesh, compiler_params=_sc_params,
        scratch_types=[
            pltpu.VMEM((NCHUNK, K), jnp.int32),
            pltpu.VMEM((NCHUNK, K), jnp.int32),
            pltpu.VMEM((K, D), jnp.float32),
            pltpu.VMEM((K, D), jnp.float32),
            pltpu.VMEM_SHARED((N, D), jnp.float32),
            pltpu.SemaphoreType.DMA,
            pltpu.SemaphoreType.DMA,
            pltpu.SemaphoreType.DMA,
            pltpu.SemaphoreType.DMA,
        ])
    def agg_kernel(h_hbm, src_hbm, dst_hbm, out_hbm,
                   srcv, dstv, bufa, bufb, accum, sema, semb, aa, ab):
        ci = lax.axis_index("c")
        si = lax.axis_index("s")
        wid = si * NC + ci

        @pl.loop(0, K)
        def _(r):
            @pl.loop(0, D, step=L)
            def _(j):
                bufa[r, pl.ds(j, L)] = jnp.zeros((L,), jnp.float32)

        @pl.loop(0, ZROUNDS)
        def _(k):
            chunk = si + NS * k

            @pl.when(chunk < N // K)
            def _():
                pltpu.async_copy(bufa, accum.at[pl.ds(chunk * K, K)], aa)

        @pl.loop(0, ZROUNDS)
        def _(k):
            chunk = si + NS * k

            @pl.when(chunk < N // K)
            def _():
                pltpu.make_async_copy(bufa, accum.at[pl.ds(chunk * K, K)], aa).wait()

        plsc.subcore_barrier()
        pltpu.sync_copy(src_hbm.at[wid], srcv)
        pltpu.sync_copy(dst_hbm.at[wid], dstv)

        pltpu.async_copy(h_hbm.at[srcv.at[0]], bufa, sema)
        pltpu.async_copy(h_hbm.at[srcv.at[1]], bufb, semb)

        @pl.loop(0, NCHUNK, step=2)
        def _(c):
            pltpu.make_async_copy(h_hbm.at[srcv.at[c]], bufa, sema).wait()
            pltpu.async_copy(bufa, accum.at[dstv.at[c]], aa, add=True)

            @pl.when(c + 1 < NCHUNK)
            def _():
                pltpu.make_async_copy(h_hbm.at[srcv.at[c + 1]], bufb, semb).wait()
                pltpu.async_copy(bufb, accum.at[dstv.at[c + 1]], ab, add=True)

            pltpu.make_async_copy(bufa, accum.at[dstv.at[c]], aa).wait()

            @pl.when(c + 2 < NCHUNK)
            def _():
                pltpu.async_copy(h_hbm.at[srcv.at[c + 2]], bufa, sema)

            @pl.when(c + 1 < NCHUNK)
            def _():
                pltpu.make_async_copy(bufb, accum.at[dstv.at[c + 1]], ab).wait()

            @pl.when(c + 3 < NCHUNK)
            def _():
                pltpu.async_copy(h_hbm.at[srcv.at[c + 3]], bufb, semb)

        plsc.subcore_barrier()

        @pl.loop(0, ZROUNDS)
        def _(k):
            chunk = si + NS * k

            @pl.when(chunk < N // K)
            def _():
                base = chunk * K
                pltpu.async_copy(accum.at[pl.ds(base, K)],
                                 out_hbm.at[ci, pl.ds(base, K)], aa)

        @pl.loop(0, ZROUNDS)
        def _(k):
            chunk = si + NS * k

            @pl.when(chunk < N // K)
            def _():
                base = chunk * K
                pltpu.make_async_copy(accum.at[pl.ds(base, K)],
                                      out_hbm.at[ci, pl.ds(base, K)], aa).wait()

    return agg_kernel(h, src3, dst3)


_R = 1000


def _norm_from(counts_ref):
    c = counts_ref[0, :, 0:1] + counts_ref[1, :, 0:1]
    return lax.rsqrt(jnp.maximum(c, 1.0))


def _scale_matmul(x, cs, W):
    D, Do = W.shape

    def body(x_ref, cs_ref, w_ref, o_ref):
        o_ref[...] = jnp.dot(x_ref[...] * _norm_from(cs_ref), w_ref[...],
                             preferred_element_type=jnp.float32)

    return pl.pallas_call(
        body, grid=(N // _R,),
        in_specs=[pl.BlockSpec((_R, D), lambda i: (i, 0)),
                  pl.BlockSpec((NC, _R, L), lambda i: (0, i, 0)),
                  pl.BlockSpec((D, Do), lambda i: (0, 0))],
        out_specs=pl.BlockSpec((_R, Do), lambda i: (i, 0)),
        out_shape=jax.ShapeDtypeStruct((N, Do), jnp.float32))(x, cs, W)


def _update_matmul(agg, cd, cs, b, W):
    D, Do = W.shape

    def body(a_ref, cd_ref, cs_ref, b_ref, w_ref, o_ref):
        a = a_ref[0] + a_ref[1]
        h = jnp.maximum(a * _norm_from(cd_ref) + b_ref[...], 0.0) \
            * _norm_from(cs_ref)
        o_ref[...] = jnp.dot(h, w_ref[...], preferred_element_type=jnp.float32)

    return pl.pallas_call(
        body, grid=(N // _R,),
        in_specs=[pl.BlockSpec((NC, _R, D), lambda i: (0, i, 0)),
                  pl.BlockSpec((NC, _R, L), lambda i: (0, i, 0)),
                  pl.BlockSpec((NC, _R, L), lambda i: (0, i, 0)),
                  pl.BlockSpec((1, D), lambda i: (0, 0)),
                  pl.BlockSpec((D, Do), lambda i: (0, 0))],
        out_specs=pl.BlockSpec((_R, Do), lambda i: (i, 0)),
        out_shape=jax.ShapeDtypeStruct((N, Do), jnp.float32))(agg, cd, cs, b, W)


def _finalize(agg, cd, b):
    D = agg.shape[2]
    Do = 40

    def body(a_ref, cd_ref, b_ref, o_ref):
        a = a_ref[0] + a_ref[1]
        o_ref[...] = (a * _norm_from(cd_ref) + b_ref[...])[:, :Do]

    return pl.pallas_call(
        body, grid=(N // _R,),
        in_specs=[pl.BlockSpec((NC, _R, D), lambda i: (0, i, 0)),
                  pl.BlockSpec((NC, _R, L), lambda i: (0, i, 0)),
                  pl.BlockSpec((1, D), lambda i: (0, 0))],
        out_specs=pl.BlockSpec((_R, Do), lambda i: (i, 0)),
        out_shape=jax.ShapeDtypeStruct((N, Do), jnp.float32))(agg, cd, b)


def kernel(features, edge_index, W1, b1, W2, b2, W3, b3):
    src = edge_index[0].astype(jnp.int32)
    src3 = src.reshape(NW, NCHUNK, K)
    dst3 = edge_index[1].astype(jnp.int32).reshape(NW, NCHUNK, K)
    srclo3 = (src * 2).reshape(NW, NCHUNK, K)
    srchi3 = srclo3 + 1

    cs, cd = _degrees(src3, dst3)

    h0 = _scale_matmul(features, cs, W1).reshape(2 * N, _D // 2)
    a1 = _aggregate_full(h0, srclo3, srchi3, dst3)
    h1 = _update_matmul(a1, cd, cs, b1.reshape(1, -1),
                        W2).reshape(2 * N, _D // 2)
    a2 = _aggregate_full(h1, srclo3, srchi3, dst3)

    W3p = jnp.pad(W3, ((0, 0), (0, 8)))
    b3p = jnp.pad(b3, (0, 8))
    h2 = _update_matmul(a2, cd, cs, b2.reshape(1, -1), W3p)
    a3 = _aggregate(h2, src3, dst3)
    out = _finalize(a3, cd, b3p.reshape(1, -1))
    return out


# speedup vs baseline: 1.1495x; 1.0654x over previous
import functools

import jax
import jax.numpy as jnp
from jax import lax
from jax.experimental import pallas as pl
from jax.experimental.pallas import tpu as pltpu
from jax.experimental.pallas import tpu_sc as plsc

N = 10000
E = 320000
NC, NS, L = 2, 16, 16
NW = NC * NS
EPW = E // NW
K = 80
NCHUNK = EPW // K
ROWS_PER_SUB = N // NS
ZROUNDS = (N // K + NS - 1) // NS

_mesh = plsc.VectorSubcoreMesh(core_axis_name="c", subcore_axis_name="s")
_sc_params = pltpu.CompilerParams(use_tc_tiling_on_sc=False)


def _degrees(src3, dst3):
    out_type = (jax.ShapeDtypeStruct((NC, N, L), jnp.float32),
                jax.ShapeDtypeStruct((NC, N, L), jnp.float32))

    @functools.partial(
        pl.kernel, out_type=out_type, mesh=_mesh, compiler_params=_sc_params,
        scratch_types=[
            pltpu.VMEM((NCHUNK, K), jnp.int32),
            pltpu.VMEM((NCHUNK, K), jnp.int32),
            pltpu.VMEM((K, L), jnp.float32),
            pltpu.VMEM((K, L), jnp.float32),
            pltpu.VMEM_SHARED((N, L), jnp.float32),
            pltpu.VMEM_SHARED((N, L), jnp.float32),
            pltpu.SemaphoreType.DMA,
            pltpu.SemaphoreType.DMA,
            pltpu.SemaphoreType.DMA,
        ])
    def deg_kernel(src_hbm, dst_hbm, os_hbm, od_hbm,
                   srcv, dstv, onesv, zerov, accs, accd, sem, ssem, dsem):
        ci = lax.axis_index("c")
        si = lax.axis_index("s")
        wid = si * NC + ci

        @pl.loop(0, K)
        def _(r):
            onesv[r, :] = jnp.ones((L,), jnp.float32)
            zerov[r, :] = jnp.zeros((L,), jnp.float32)

        @pl.loop(0, ZROUNDS)
        def _(k):
            chunk = si + NS * k

            @pl.when(chunk < N // K)
            def _():
                pltpu.async_copy(zerov, accs.at[pl.ds(chunk * K, K)], sem)
                pltpu.async_copy(zerov, accd.at[pl.ds(chunk * K, K)], sem)

        @pl.loop(0, ZROUNDS)
        def _(k):
            chunk = si + NS * k

            @pl.when(chunk < N // K)
            def _():
                pltpu.make_async_copy(zerov, accs.at[pl.ds(chunk * K, K)], sem).wait()
                pltpu.make_async_copy(zerov, accd.at[pl.ds(chunk * K, K)], sem).wait()

        plsc.subcore_barrier()
        pltpu.sync_copy(src_hbm.at[wid], srcv)
        pltpu.sync_copy(dst_hbm.at[wid], dstv)

        @pl.loop(0, NCHUNK)
        def _(c):
            pltpu.async_copy(onesv, accs.at[srcv.at[c]], ssem, add=True)
            pltpu.async_copy(onesv, accd.at[dstv.at[c]], dsem, add=True)
            pltpu.make_async_copy(onesv, accs.at[srcv.at[c]], ssem).wait()
            pltpu.make_async_copy(onesv, accd.at[dstv.at[c]], dsem).wait()

        plsc.subcore_barrier()

        @pl.loop(0, ZROUNDS)
        def _(k):
            chunk = si + NS * k

            @pl.when(chunk < N // K)
            def _():
                base = chunk * K
                pltpu.async_copy(accs.at[pl.ds(base, K)],
                                 os_hbm.at[ci, pl.ds(base, K)], sem)
                pltpu.async_copy(accd.at[pl.ds(base, K)],
                                 od_hbm.at[ci, pl.ds(base, K)], sem)

        @pl.loop(0, ZROUNDS)
        def _(k):
            chunk = si + NS * k

            @pl.when(chunk < N // K)
            def _():
                base = chunk * K
                pltpu.make_async_copy(accs.at[pl.ds(base, K)],
                                      os_hbm.at[ci, pl.ds(base, K)], sem).wait()
                pltpu.make_async_copy(accd.at[pl.ds(base, K)],
                                      od_hbm.at[ci, pl.ds(base, K)], sem).wait()

    return deg_kernel(src3, dst3)


_SEC = 25
_NSEC = NCHUNK // _SEC
_D = 128


def _aggregate_full(h2, srclo3, srchi3, dst3):
    H = _D // 2
    out_type = jax.ShapeDtypeStruct((NC, N, _D), jnp.float32)

    @functools.partial(
        pl.kernel, out_type=out_type, mesh=_mesh, compiler_params=_sc_params,
        scratch_types=[
            pltpu.VMEM((_SEC, K), jnp.int32),
            pltpu.VMEM((_SEC, K), jnp.int32),
            pltpu.VMEM((_SEC, K), jnp.int32),
            pltpu.VMEM((K, H), jnp.float32),
            pltpu.VMEM((K, H), jnp.float32),
            pltpu.VMEM((K, H), jnp.float32),
            pltpu.VMEM((K, H), jnp.float32),
            pltpu.VMEM((K, H), jnp.float32),
            pltpu.VMEM((K, H), jnp.float32),
            pltpu.VMEM_SHARED((N, H), jnp.float32),
            pltpu.VMEM_SHARED((N, H), jnp.float32),
            pltpu.SemaphoreType.DMA,
            pltpu.SemaphoreType.DMA,
            pltpu.SemaphoreType.DMA,
            pltpu.SemaphoreType.DMA,
            pltpu.SemaphoreType.DMA,
            pltpu.SemaphoreType.DMA,
            pltpu.SemaphoreType.DMA,
            pltpu.SemaphoreType.DMA,
            pltpu.SemaphoreType.DMA,
            pltpu.SemaphoreType.DMA,
            pltpu.SemaphoreType.DMA,
            pltpu.SemaphoreType.DMA,
        ])
    def agg2_kernel(h_hbm, srclo_hbm, srchi_hbm, dst_hbm, out_hbm,
                    srclov, srchiv, dstv, la, lb, lc, ha, hb, hc,
                    acc_lo, acc_hi,
                    sla, slb, slc, sha, shb, shc,
                    ala, alb, alc, aha, ahb, ahc):
        ci = lax.axis_index("c")
        si = lax.axis_index("s")
        wid = si * NC + ci

        @pl.loop(0, K)
        def _(r):
            @pl.loop(0, H, step=L)
            def _(j):
                la[r, pl.ds(j, L)] = jnp.zeros((L,), jnp.float32)

        @pl.loop(0, ZROUNDS)
        def _(k):
            chunk = si + NS * k

            @pl.when(chunk < N // K)
            def _():
                pltpu.async_copy(la, acc_lo.at[pl.ds(chunk * K, K)], ala)
                pltpu.async_copy(la, acc_hi.at[pl.ds(chunk * K, K)], aha)

        @pl.loop(0, ZROUNDS)
        def _(k):
            chunk = si + NS * k

            @pl.when(chunk < N // K)
            def _():
                pltpu.make_async_copy(la, acc_lo.at[pl.ds(chunk * K, K)], ala).wait()
                pltpu.make_async_copy(la, acc_hi.at[pl.ds(chunk * K, K)], aha).wait()

        plsc.subcore_barrier()

        @pl.loop(0, _NSEC)
        def _(s):
            pltpu.sync_copy(srclo_hbm.at[wid, pl.ds(s * _SEC, _SEC)], srclov)
            pltpu.sync_copy(srchi_hbm.at[wid, pl.ds(s * _SEC, _SEC)], srchiv)
            pltpu.sync_copy(dst_hbm.at[wid, pl.ds(s * _SEC, _SEC)], dstv)

            pltpu.async_copy(h_hbm.at[srclov.at[0]], la, sla)
            pltpu.async_copy(h_hbm.at[srchiv.at[0]], ha, sha)
            pltpu.async_copy(h_hbm.at[srclov.at[1]], lb, slb)
            pltpu.async_copy(h_hbm.at[srchiv.at[1]], hb, shb)
            pltpu.async_copy(h_hbm.at[srclov.at[2]], lc, slc)
            pltpu.async_copy(h_hbm.at[srchiv.at[2]], hc, shc)

            @pl.loop(0, _SEC, step=3)
            def _(c):
                pltpu.make_async_copy(h_hbm.at[srclov.at[c]], la, sla).wait()
                pltpu.async_copy(la, acc_lo.at[dstv.at[c]], ala, add=True)
                pltpu.make_async_copy(h_hbm.at[srchiv.at[c]], ha, sha).wait()
                pltpu.async_copy(ha, acc_hi.at[dstv.at[c]], aha, add=True)

                @pl.when(c + 1 < _SEC)
                def _():
                    pltpu.make_async_copy(h_hbm.at[srclov.at[c + 1]], lb, slb).wait()
                    pltpu.async_copy(lb, acc_lo.at[dstv.at[c + 1]], alb, add=True)
                    pltpu.make_async_copy(h_hbm.at[srchiv.at[c + 1]], hb, shb).wait()
                    pltpu.async_copy(hb, acc_hi.at[dstv.at[c + 1]], ahb, add=True)

                pltpu.make_async_copy(la, acc_lo.at[dstv.at[c]], ala).wait()
                pltpu.make_async_copy(ha, acc_hi.at[dstv.at[c]], aha).wait()

                @pl.when(c + 3 < _SEC)
                def _():
                    pltpu.async_copy(h_hbm.at[srclov.at[c + 3]], la, sla)
                    pltpu.async_copy(h_hbm.at[srchiv.at[c + 3]], ha, sha)

                @pl.when(c + 2 < _SEC)
                def _():
                    pltpu.make_async_copy(h_hbm.at[srclov.at[c + 2]], lc, slc).wait()
                    pltpu.async_copy(lc, acc_lo.at[dstv.at[c + 2]], alc, add=True)
                    pltpu.make_async_copy(h_hbm.at[srchiv.at[c + 2]], hc, shc).wait()
                    pltpu.async_copy(hc, acc_hi.at[dstv.at[c + 2]], ahc, add=True)

                @pl.when(c + 1 < _SEC)
                def _():
                    pltpu.make_async_copy(lb, acc_lo.at[dstv.at[c + 1]], alb).wait()
                    pltpu.make_async_copy(hb, acc_hi.at[dstv.at[c + 1]], ahb).wait()

                @pl.when(c + 4 < _SEC)
                def _():
                    pltpu.async_copy(h_hbm.at[srclov.at[c + 4]], lb, slb)
                    pltpu.async_copy(h_hbm.at[srchiv.at[c + 4]], hb, shb)

                @pl.when(c + 2 < _SEC)
                def _():
                    pltpu.make_async_copy(lc, acc_lo.at[dstv.at[c + 2]], alc).wait()
                    pltpu.make_async_copy(hc, acc_hi.at[dstv.at[c + 2]], ahc).wait()

                @pl.when(c + 5 < _SEC)
                def _():
                    pltpu.async_copy(h_hbm.at[srclov.at[c + 5]], lc, slc)
                    pltpu.async_copy(h_hbm.at[srchiv.at[c + 5]], hc, shc)

        plsc.subcore_barrier()

        @pl.loop(0, ZROUNDS)
        def _(k):
            chunk = si + NS * k

            @pl.when(chunk < N // K)
            def _():
                base = chunk * K
                pltpu.async_copy(acc_lo.at[pl.ds(base, K)],
                                 out_hbm.at[ci, pl.ds(base, K), pl.ds(0, H)], ala)
                pltpu.async_copy(acc_hi.at[pl.ds(base, K)],
                                 out_hbm.at[ci, pl.ds(base, K), pl.ds(H, H)], aha)

        @pl.loop(0, ZROUNDS)
        def _(k):
            chunk = si + NS * k

            @pl.when(chunk < N // K)
            def _():
                base = chunk * K
                pltpu.make_async_copy(
                    acc_lo.at[pl.ds(base, K)],
                    out_hbm.at[ci, pl.ds(base, K), pl.ds(0, H)], ala).wait()
                pltpu.make_async_copy(
                    acc_hi.at[pl.ds(base, K)],
                    out_hbm.at[ci, pl.ds(base, K), pl.ds(H, H)], aha).wait()

    return agg2_kernel(h2, srclo3, srchi3, dst3)


def _aggregate(h, src3, dst3):
    D = h.shape[1]

    @functools.partial(
        pl.kernel, out_type=jax.ShapeDtypeStruct((NC, N, D), jnp.float32),
        mesh=_mesh, compiler_params=_sc_params,
        scratch_types=[
            pltpu.VMEM((NCHUNK, K), jnp.int32),
            pltpu.VMEM((NCHUNK, K), jnp.int32),
            pltpu.VMEM((K, D), jnp.float32),
            pltpu.VMEM((K, D), jnp.float32),
            pltpu.VMEM_SHARED((N, D), jnp.float32),
            pltpu.SemaphoreType.DMA,
            pltpu.SemaphoreType.DMA,
            pltpu.SemaphoreType.DMA,
            pltpu.SemaphoreType.DMA,
        ])
    def agg_kernel(h_hbm, src_hbm, dst_hbm, out_hbm,
                   srcv, dstv, bufa, bufb, accum, sema, semb, aa, ab):
        ci = lax.axis_index("c")
        si = lax.axis_index("s")
        wid = si * NC + ci

        @pl.loop(0, K)
        def _(r):
            @pl.loop(0, D, step=L)
            def _(j):
                bufa[r, pl.ds(j, L)] = jnp.zeros((L,), jnp.float32)

        @pl.loop(0, ZROUNDS)
        def _(k):
            chunk = si + NS * k

            @pl.when(chunk < N // K)
            def _():
                pltpu.async_copy(bufa, accum.at[pl.ds(chunk * K, K)], aa)

        @pl.loop(0, ZROUNDS)
        def _(k):
            chunk = si + NS * k

            @pl.when(chunk < N // K)
            def _():
                pltpu.make_async_copy(bufa, accum.at[pl.ds(chunk * K, K)], aa).wait()

        plsc.subcore_barrier()
        pltpu.sync_copy(src_hbm.at[wid], srcv)
        pltpu.sync_copy(dst_hbm.at[wid], dstv)

        pltpu.async_copy(h_hbm.at[srcv.at[0]], bufa, sema)
        pltpu.async_copy(h_hbm.at[srcv.at[1]], bufb, semb)

        @pl.loop(0, NCHUNK, step=2)
        def _(c):
            pltpu.make_async_copy(h_hbm.at[srcv.at[c]], bufa, sema).wait()
            pltpu.async_copy(bufa, accum.at[dstv.at[c]], aa, add=True)

            @pl.when(c + 1 < NCHUNK)
            def _():
                pltpu.make_async_copy(h_hbm.at[srcv.at[c + 1]], bufb, semb).wait()
                pltpu.async_copy(bufb, accum.at[dstv.at[c + 1]], ab, add=True)

            pltpu.make_async_copy(bufa, accum.at[dstv.at[c]], aa).wait()

            @pl.when(c + 2 < NCHUNK)
            def _():
                pltpu.async_copy(h_hbm.at[srcv.at[c + 2]], bufa, sema)

            @pl.when(c + 1 < NCHUNK)
            def _():
                pltpu.make_async_copy(bufb, accum.at[dstv.at[c + 1]], ab).wait()

            @pl.when(c + 3 < NCHUNK)
            def _():
                pltpu.async_copy(h_hbm.at[srcv.at[c + 3]], bufb, semb)

        plsc.subcore_barrier()

        @pl.loop(0, ZROUNDS)
        def _(k):
            chunk = si + NS * k

            @pl.when(chunk < N // K)
            def _():
                base = chunk * K
                pltpu.async_copy(accum.at[pl.ds(base, K)],
                                 out_hbm.at[ci, pl.ds(base, K)], aa)

        @pl.loop(0, ZROUNDS)
        def _(k):
            chunk = si + NS * k

            @pl.when(chunk < N // K)
            def _():
                base = chunk * K
                pltpu.make_async_copy(accum.at[pl.ds(base, K)],
                                      out_hbm.at[ci, pl.ds(base, K)], aa).wait()

    return agg_kernel(h, src3, dst3)


_R = 1000


def _norm_from(counts_ref):
    c = counts_ref[0, :, 0:1] + counts_ref[1, :, 0:1]
    return lax.rsqrt(jnp.maximum(c, 1.0))


def _scale_matmul(x, cs, W):
    D, Do = W.shape

    def body(x_ref, cs_ref, w_ref, o_ref):
        o_ref[...] = jnp.dot(x_ref[...] * _norm_from(cs_ref), w_ref[...],
                             preferred_element_type=jnp.float32)

    return pl.pallas_call(
        body, grid=(N // _R,),
        in_specs=[pl.BlockSpec((_R, D), lambda i: (i, 0)),
                  pl.BlockSpec((NC, _R, L), lambda i: (0, i, 0)),
                  pl.BlockSpec((D, Do), lambda i: (0, 0))],
        out_specs=pl.BlockSpec((_R, Do), lambda i: (i, 0)),
        out_shape=jax.ShapeDtypeStruct((N, Do), jnp.float32))(x, cs, W)


def _update_matmul(agg, cd, cs, b, W):
    D, Do = W.shape

    def body(a_ref, cd_ref, cs_ref, b_ref, w_ref, o_ref):
        a = a_ref[0] + a_ref[1]
        h = jnp.maximum(a * _norm_from(cd_ref) + b_ref[...], 0.0) \
            * _norm_from(cs_ref)
        o_ref[...] = jnp.dot(h, w_ref[...], preferred_element_type=jnp.float32)

    return pl.pallas_call(
        body, grid=(N // _R,),
        in_specs=[pl.BlockSpec((NC, _R, D), lambda i: (0, i, 0)),
                  pl.BlockSpec((NC, _R, L), lambda i: (0, i, 0)),
                  pl.BlockSpec((NC, _R, L), lambda i: (0, i, 0)),
                  pl.BlockSpec((1, D), lambda i: (0, 0)),
                  pl.BlockSpec((D, Do), lambda i: (0, 0))],
        out_specs=pl.BlockSpec((_R, Do), lambda i: (i, 0)),
        out_shape=jax.ShapeDtypeStruct((N, Do), jnp.float32))(agg, cd, cs, b, W)


def _finalize(agg, cd, b):
    D = agg.shape[2]
    Do = 40

    def body(a_ref, cd_ref, b_ref, o_ref):
        a = a_ref[0] + a_ref[1]
        o_ref[...] = (a * _norm_from(cd_ref) + b_ref[...])[:, :Do]

    return pl.pallas_call(
        body, grid=(N // _R,),
        in_specs=[pl.BlockSpec((NC, _R, D), lambda i: (0, i, 0)),
                  pl.BlockSpec((NC, _R, L), lambda i: (0, i, 0)),
                  pl.BlockSpec((1, D), lambda i: (0, 0))],
        out_specs=pl.BlockSpec((_R, Do), lambda i: (i, 0)),
        out_shape=jax.ShapeDtypeStruct((N, Do), jnp.float32))(agg, cd, b)


def kernel(features, edge_index, W1, b1, W2, b2, W3, b3):
    src = edge_index[0].astype(jnp.int32)
    src3 = src.reshape(NW, NCHUNK, K)
    dst3 = edge_index[1].astype(jnp.int32).reshape(NW, NCHUNK, K)
    srclo3 = (src * 2).reshape(NW, NCHUNK, K)
    srchi3 = srclo3 + 1

    cs, cd = _degrees(src3, dst3)

    h0 = _scale_matmul(features, cs, W1).reshape(2 * N, _D // 2)
    a1 = _aggregate_full(h0, srclo3, srchi3, dst3)
    h1 = _update_matmul(a1, cd, cs, b1.reshape(1, -1),
                        W2).reshape(2 * N, _D // 2)
    a2 = _aggregate_full(h1, srclo3, srchi3, dst3)

    W3p = jnp.pad(W3, ((0, 0), (0, 8)))
    b3p = jnp.pad(b3, (0, 8))
    h2 = _update_matmul(a2, cd, cs, b2.reshape(1, -1), W3p)
    a3 = _aggregate(h2, src3, dst3)
    out = _finalize(a3, cd, b3p.reshape(1, -1))
    return out


# speedup vs baseline: 1.2027x; 1.0463x over previous
import functools

import jax
import jax.numpy as jnp
from jax import lax
from jax.experimental import pallas as pl
from jax.experimental.pallas import tpu as pltpu
from jax.experimental.pallas import tpu_sc as plsc

N = 10000
E = 320000
NC, NS, L = 2, 16, 16
NW = NC * NS
EPW = E // NW
K = 80
NCHUNK = EPW // K
ROWS_PER_SUB = N // NS
ZROUNDS = (N // K + NS - 1) // NS

_mesh = plsc.VectorSubcoreMesh(core_axis_name="c", subcore_axis_name="s")
_sc_params = pltpu.CompilerParams(use_tc_tiling_on_sc=False)


def _degrees(src3, dst3):
    out_type = (jax.ShapeDtypeStruct((NC, N, L), jnp.float32),
                jax.ShapeDtypeStruct((NC, N, L), jnp.float32))

    @functools.partial(
        pl.kernel, out_type=out_type, mesh=_mesh, compiler_params=_sc_params,
        scratch_types=[
            pltpu.VMEM((NCHUNK, K), jnp.int32),
            pltpu.VMEM((NCHUNK, K), jnp.int32),
            pltpu.VMEM((K, L), jnp.float32),
            pltpu.VMEM((K, L), jnp.float32),
            pltpu.VMEM_SHARED((N, L), jnp.float32),
            pltpu.VMEM_SHARED((N, L), jnp.float32),
            pltpu.SemaphoreType.DMA,
            pltpu.SemaphoreType.DMA,
            pltpu.SemaphoreType.DMA,
        ])
    def deg_kernel(src_hbm, dst_hbm, os_hbm, od_hbm,
                   srcv, dstv, onesv, zerov, accs, accd, sem, ssem, dsem):
        ci = lax.axis_index("c")
        si = lax.axis_index("s")
        wid = si * NC + ci

        @pl.loop(0, K)
        def _(r):
            onesv[r, :] = jnp.ones((L,), jnp.float32)
            zerov[r, :] = jnp.zeros((L,), jnp.float32)

        @pl.loop(0, ZROUNDS)
        def _(k):
            chunk = si + NS * k

            @pl.when(chunk < N // K)
            def _():
                pltpu.async_copy(zerov, accs.at[pl.ds(chunk * K, K)], sem)
                pltpu.async_copy(zerov, accd.at[pl.ds(chunk * K, K)], sem)

        @pl.loop(0, ZROUNDS)
        def _(k):
            chunk = si + NS * k

            @pl.when(chunk < N // K)
            def _():
                pltpu.make_async_copy(zerov, accs.at[pl.ds(chunk * K, K)], sem).wait()
                pltpu.make_async_copy(zerov, accd.at[pl.ds(chunk * K, K)], sem).wait()

        plsc.subcore_barrier()
        pltpu.sync_copy(src_hbm.at[wid], srcv)
        pltpu.sync_copy(dst_hbm.at[wid], dstv)

        @pl.loop(0, NCHUNK)
        def _(c):
            pltpu.async_copy(onesv, accs.at[srcv.at[c]], ssem, add=True)
            pltpu.async_copy(onesv, accd.at[dstv.at[c]], dsem, add=True)
            pltpu.make_async_copy(onesv, accs.at[srcv.at[c]], ssem).wait()
            pltpu.make_async_copy(onesv, accd.at[dstv.at[c]], dsem).wait()

        plsc.subcore_barrier()

        @pl.loop(0, ZROUNDS)
        def _(k):
            chunk = si + NS * k

            @pl.when(chunk < N // K)
            def _():
                base = chunk * K
                pltpu.async_copy(accs.at[pl.ds(base, K)],
                                 os_hbm.at[ci, pl.ds(base, K)], sem)
                pltpu.async_copy(accd.at[pl.ds(base, K)],
                                 od_hbm.at[ci, pl.ds(base, K)], sem)

        @pl.loop(0, ZROUNDS)
        def _(k):
            chunk = si + NS * k

            @pl.when(chunk < N // K)
            def _():
                base = chunk * K
                pltpu.make_async_copy(accs.at[pl.ds(base, K)],
                                      os_hbm.at[ci, pl.ds(base, K)], sem).wait()
                pltpu.make_async_copy(accd.at[pl.ds(base, K)],
                                      od_hbm.at[ci, pl.ds(base, K)], sem).wait()

    return deg_kernel(src3, dst3)


_SEC = 25
_NSEC = NCHUNK // _SEC
_D = 128


def _aggregate_full(h2, srclo3, srchi3, dst3):
    H = _D // 2
    out_type = jax.ShapeDtypeStruct((NC, N, _D), jnp.float32)

    @functools.partial(
        pl.kernel, out_type=out_type, mesh=_mesh, compiler_params=_sc_params,
        scratch_types=[
            pltpu.VMEM((_SEC, K), jnp.int32),
            pltpu.VMEM((_SEC, K), jnp.int32),
            pltpu.VMEM((_SEC, K), jnp.int32),
            pltpu.VMEM((K, H), jnp.float32),
            pltpu.VMEM((K, H), jnp.float32),
            pltpu.VMEM((K, H), jnp.float32),
            pltpu.VMEM((K, H), jnp.float32),
            pltpu.VMEM((K, H), jnp.float32),
            pltpu.VMEM((K, H), jnp.float32),
            pltpu.VMEM_SHARED((N, H), jnp.float32),
            pltpu.VMEM_SHARED((N, H), jnp.float32),
            pltpu.SemaphoreType.DMA,
            pltpu.SemaphoreType.DMA,
            pltpu.SemaphoreType.DMA,
            pltpu.SemaphoreType.DMA,
            pltpu.SemaphoreType.DMA,
            pltpu.SemaphoreType.DMA,
            pltpu.SemaphoreType.DMA,
            pltpu.SemaphoreType.DMA,
            pltpu.SemaphoreType.DMA,
            pltpu.SemaphoreType.DMA,
            pltpu.SemaphoreType.DMA,
            pltpu.SemaphoreType.DMA,
        ])
    def agg2_kernel(h_hbm, srclo_hbm, srchi_hbm, dst_hbm, out_hbm,
                    srclov, srchiv, dstv, la, lb, lc, ha, hb, hc,
                    acc_lo, acc_hi,
                    sla, slb, slc, sha, shb, shc,
                    ala, alb, alc, aha, ahb, ahc):
        ci = lax.axis_index("c")
        si = lax.axis_index("s")
        wid = si * NC + ci

        @pl.loop(0, K)
        def _(r):
            @pl.loop(0, H, step=L)
            def _(j):
                la[r, pl.ds(j, L)] = jnp.zeros((L,), jnp.float32)

        @pl.loop(0, ZROUNDS)
        def _(k):
            chunk = si + NS * k

            @pl.when(chunk < N // K)
            def _():
                pltpu.async_copy(la, acc_lo.at[pl.ds(chunk * K, K)], ala)
                pltpu.async_copy(la, acc_hi.at[pl.ds(chunk * K, K)], aha)

        @pl.loop(0, ZROUNDS)
        def _(k):
            chunk = si + NS * k

            @pl.when(chunk < N // K)
            def _():
                pltpu.make_async_copy(la, acc_lo.at[pl.ds(chunk * K, K)], ala).wait()
                pltpu.make_async_copy(la, acc_hi.at[pl.ds(chunk * K, K)], aha).wait()

        plsc.subcore_barrier()

        @pl.loop(0, _NSEC)
        def _(s):
            pltpu.sync_copy(srclo_hbm.at[wid, pl.ds(s * _SEC, _SEC)], srclov)
            pltpu.sync_copy(srchi_hbm.at[wid, pl.ds(s * _SEC, _SEC)], srchiv)
            pltpu.sync_copy(dst_hbm.at[wid, pl.ds(s * _SEC, _SEC)], dstv)

            pltpu.async_copy(h_hbm.at[srclov.at[0]], la, sla)
            pltpu.async_copy(h_hbm.at[srchiv.at[0]], ha, sha)
            pltpu.async_copy(h_hbm.at[srclov.at[1]], lb, slb)
            pltpu.async_copy(h_hbm.at[srchiv.at[1]], hb, shb)
            pltpu.async_copy(h_hbm.at[srclov.at[2]], lc, slc)
            pltpu.async_copy(h_hbm.at[srchiv.at[2]], hc, shc)

            @pl.loop(0, _SEC, step=3)
            def _(c):
                pltpu.make_async_copy(h_hbm.at[srclov.at[c]], la, sla).wait()
                pltpu.async_copy(la, acc_lo.at[dstv.at[c]], ala, add=True)
                pltpu.make_async_copy(h_hbm.at[srchiv.at[c]], ha, sha).wait()
                pltpu.async_copy(ha, acc_hi.at[dstv.at[c]], aha, add=True)

                @pl.when(c + 1 < _SEC)
                def _():
                    pltpu.make_async_copy(h_hbm.at[srclov.at[c + 1]], lb, slb).wait()
                    pltpu.async_copy(lb, acc_lo.at[dstv.at[c + 1]], alb, add=True)
                    pltpu.make_async_copy(h_hbm.at[srchiv.at[c + 1]], hb, shb).wait()
                    pltpu.async_copy(hb, acc_hi.at[dstv.at[c + 1]], ahb, add=True)

                pltpu.make_async_copy(la, acc_lo.at[dstv.at[c]], ala).wait()
                pltpu.make_async_copy(ha, acc_hi.at[dstv.at[c]], aha).wait()

                @pl.when(c + 3 < _SEC)
                def _():
                    pltpu.async_copy(h_hbm.at[srclov.at[c + 3]], la, sla)
                    pltpu.async_copy(h_hbm.at[srchiv.at[c + 3]], ha, sha)

                @pl.when(c + 2 < _SEC)
                def _():
                    pltpu.make_async_copy(h_hbm.at[srclov.at[c + 2]], lc, slc).wait()
                    pltpu.async_copy(lc, acc_lo.at[dstv.at[c + 2]], alc, add=True)
                    pltpu.make_async_copy(h_hbm.at[srchiv.at[c + 2]], hc, shc).wait()
                    pltpu.async_copy(hc, acc_hi.at[dstv.at[c + 2]], ahc, add=True)

                @pl.when(c + 1 < _SEC)
                def _():
                    pltpu.make_async_copy(lb, acc_lo.at[dstv.at[c + 1]], alb).wait()
                    pltpu.make_async_copy(hb, acc_hi.at[dstv.at[c + 1]], ahb).wait()

                @pl.when(c + 4 < _SEC)
                def _():
                    pltpu.async_copy(h_hbm.at[srclov.at[c + 4]], lb, slb)
                    pltpu.async_copy(h_hbm.at[srchiv.at[c + 4]], hb, shb)

                @pl.when(c + 2 < _SEC)
                def _():
                    pltpu.make_async_copy(lc, acc_lo.at[dstv.at[c + 2]], alc).wait()
                    pltpu.make_async_copy(hc, acc_hi.at[dstv.at[c + 2]], ahc).wait()

                @pl.when(c + 5 < _SEC)
                def _():
                    pltpu.async_copy(h_hbm.at[srclov.at[c + 5]], lc, slc)
                    pltpu.async_copy(h_hbm.at[srchiv.at[c + 5]], hc, shc)

        plsc.subcore_barrier()

        @pl.loop(0, ZROUNDS)
        def _(k):
            chunk = si + NS * k

            @pl.when(chunk < N // K)
            def _():
                base = chunk * K
                pltpu.async_copy(acc_lo.at[pl.ds(base, K)],
                                 out_hbm.at[ci, pl.ds(base, K), pl.ds(0, H)], ala)
                pltpu.async_copy(acc_hi.at[pl.ds(base, K)],
                                 out_hbm.at[ci, pl.ds(base, K), pl.ds(H, H)], aha)

        @pl.loop(0, ZROUNDS)
        def _(k):
            chunk = si + NS * k

            @pl.when(chunk < N // K)
            def _():
                base = chunk * K
                pltpu.make_async_copy(
                    acc_lo.at[pl.ds(base, K)],
                    out_hbm.at[ci, pl.ds(base, K), pl.ds(0, H)], ala).wait()
                pltpu.make_async_copy(
                    acc_hi.at[pl.ds(base, K)],
                    out_hbm.at[ci, pl.ds(base, K), pl.ds(H, H)], aha).wait()

    return agg2_kernel(h2, srclo3, srchi3, dst3)


def _aggregate(h, src3, dst3):
    D = h.shape[1]

    @functools.partial(
        pl.kernel, out_type=jax.ShapeDtypeStruct((NC, N, D), jnp.float32),
        mesh=_mesh, compiler_params=_sc_params,
        scratch_types=[
            pltpu.VMEM((NCHUNK, K), jnp.int32),
            pltpu.VMEM((NCHUNK, K), jnp.int32),
            pltpu.VMEM((K, D), jnp.float32),
            pltpu.VMEM((K, D), jnp.float32),
            pltpu.VMEM((K, D), jnp.float32),
            pltpu.VMEM_SHARED((N, D), jnp.float32),
            pltpu.SemaphoreType.DMA,
            pltpu.SemaphoreType.DMA,
            pltpu.SemaphoreType.DMA,
            pltpu.SemaphoreType.DMA,
            pltpu.SemaphoreType.DMA,
            pltpu.SemaphoreType.DMA,
        ])
    def agg_kernel(h_hbm, src_hbm, dst_hbm, out_hbm,
                   srcv, dstv, bufa, bufb, bufc, accum,
                   sema, semb, semc, aa, ab, ac):
        ci = lax.axis_index("c")
        si = lax.axis_index("s")
        wid = si * NC + ci

        @pl.loop(0, K)
        def _(r):
            @pl.loop(0, D, step=L)
            def _(j):
                bufa[r, pl.ds(j, L)] = jnp.zeros((L,), jnp.float32)

        @pl.loop(0, ZROUNDS)
        def _(k):
            chunk = si + NS * k

            @pl.when(chunk < N // K)
            def _():
                pltpu.async_copy(bufa, accum.at[pl.ds(chunk * K, K)], aa)

        @pl.loop(0, ZROUNDS)
        def _(k):
            chunk = si + NS * k

            @pl.when(chunk < N // K)
            def _():
                pltpu.make_async_copy(bufa, accum.at[pl.ds(chunk * K, K)], aa).wait()

        plsc.subcore_barrier()
        pltpu.sync_copy(src_hbm.at[wid], srcv)
        pltpu.sync_copy(dst_hbm.at[wid], dstv)

        pltpu.async_copy(h_hbm.at[srcv.at[0]], bufa, sema)
        pltpu.async_copy(h_hbm.at[srcv.at[1]], bufb, semb)
        pltpu.async_copy(h_hbm.at[srcv.at[2]], bufc, semc)

        @pl.loop(0, NCHUNK, step=3)
        def _(c):
            pltpu.make_async_copy(h_hbm.at[srcv.at[c]], bufa, sema).wait()
            pltpu.async_copy(bufa, accum.at[dstv.at[c]], aa, add=True)

            @pl.when(c + 1 < NCHUNK)
            def _():
                pltpu.make_async_copy(h_hbm.at[srcv.at[c + 1]], bufb, semb).wait()
                pltpu.async_copy(bufb, accum.at[dstv.at[c + 1]], ab, add=True)

            pltpu.make_async_copy(bufa, accum.at[dstv.at[c]], aa).wait()

            @pl.when(c + 3 < NCHUNK)
            def _():
                pltpu.async_copy(h_hbm.at[srcv.at[c + 3]], bufa, sema)

            @pl.when(c + 2 < NCHUNK)
            def _():
                pltpu.make_async_copy(h_hbm.at[srcv.at[c + 2]], bufc, semc).wait()
                pltpu.async_copy(bufc, accum.at[dstv.at[c + 2]], ac, add=True)

            @pl.when(c + 1 < NCHUNK)
            def _():
                pltpu.make_async_copy(bufb, accum.at[dstv.at[c + 1]], ab).wait()

            @pl.when(c + 4 < NCHUNK)
            def _():
                pltpu.async_copy(h_hbm.at[srcv.at[c + 4]], bufb, semb)

            @pl.when(c + 2 < NCHUNK)
            def _():
                pltpu.make_async_copy(bufc, accum.at[dstv.at[c + 2]], ac).wait()

            @pl.when(c + 5 < NCHUNK)
            def _():
                pltpu.async_copy(h_hbm.at[srcv.at[c + 5]], bufc, semc)

        plsc.subcore_barrier()

        @pl.loop(0, ZROUNDS)
        def _(k):
            chunk = si + NS * k

            @pl.when(chunk < N // K)
            def _():
                base = chunk * K
                pltpu.async_copy(accum.at[pl.ds(base, K)],
                                 out_hbm.at[ci, pl.ds(base, K)], aa)

        @pl.loop(0, ZROUNDS)
        def _(k):
            chunk = si + NS * k

            @pl.when(chunk < N // K)
            def _():
                base = chunk * K
                pltpu.make_async_copy(accum.at[pl.ds(base, K)],
                                      out_hbm.at[ci, pl.ds(base, K)], aa).wait()

    return agg_kernel(h, src3, dst3)


_R = 1000


def _norm_from(counts_ref):
    c = counts_ref[0, :, 0:1] + counts_ref[1, :, 0:1]
    return lax.rsqrt(jnp.maximum(c, 1.0))


def _scale_matmul(x, cs, W):
    D, Do = W.shape

    def body(x_ref, cs_ref, w_ref, o_ref):
        o_ref[...] = jnp.dot(x_ref[...] * _norm_from(cs_ref), w_ref[...],
                             preferred_element_type=jnp.float32)

    return pl.pallas_call(
        body, grid=(N // _R,),
        in_specs=[pl.BlockSpec((_R, D), lambda i: (i, 0)),
                  pl.BlockSpec((NC, _R, L), lambda i: (0, i, 0)),
                  pl.BlockSpec((D, Do), lambda i: (0, 0))],
        out_specs=pl.BlockSpec((_R, Do), lambda i: (i, 0)),
        out_shape=jax.ShapeDtypeStruct((N, Do), jnp.float32))(x, cs, W)


def _update_matmul(agg, cd, cs, b, W):
    D, Do = W.shape

    def body(a_ref, cd_ref, cs_ref, b_ref, w_ref, o_ref):
        a = a_ref[0] + a_ref[1]
        h = jnp.maximum(a * _norm_from(cd_ref) + b_ref[...], 0.0) \
            * _norm_from(cs_ref)
        o_ref[...] = jnp.dot(h, w_ref[...], preferred_element_type=jnp.float32)

    return pl.pallas_call(
        body, grid=(N // _R,),
        in_specs=[pl.BlockSpec((NC, _R, D), lambda i: (0, i, 0)),
                  pl.BlockSpec((NC, _R, L), lambda i: (0, i, 0)),
                  pl.BlockSpec((NC, _R, L), lambda i: (0, i, 0)),
                  pl.BlockSpec((1, D), lambda i: (0, 0)),
                  pl.BlockSpec((D, Do), lambda i: (0, 0))],
        out_specs=pl.BlockSpec((_R, Do), lambda i: (i, 0)),
        out_shape=jax.ShapeDtypeStruct((N, Do), jnp.float32))(agg, cd, cs, b, W)


def _finalize(agg, cd, b):
    D = agg.shape[2]
    Do = 40

    def body(a_ref, cd_ref, b_ref, o_ref):
        a = a_ref[0] + a_ref[1]
        o_ref[...] = (a * _norm_from(cd_ref) + b_ref[...])[:, :Do]

    return pl.pallas_call(
        body, grid=(N // _R,),
        in_specs=[pl.BlockSpec((NC, _R, D), lambda i: (0, i, 0)),
                  pl.BlockSpec((NC, _R, L), lambda i: (0, i, 0)),
                  pl.BlockSpec((1, D), lambda i: (0, 0))],
        out_specs=pl.BlockSpec((_R, Do), lambda i: (i, 0)),
        out_shape=jax.ShapeDtypeStruct((N, Do), jnp.float32))(agg, cd, b)


def kernel(features, edge_index, W1, b1, W2, b2, W3, b3):
    src = edge_index[0].astype(jnp.int32)
    src3 = src.reshape(NW, NCHUNK, K)
    dst3 = edge_index[1].astype(jnp.int32).reshape(NW, NCHUNK, K)
    srclo3 = (src * 2).reshape(NW, NCHUNK, K)
    srchi3 = srclo3 + 1

    cs, cd = _degrees(src3, dst3)

    h0 = _scale_matmul(features, cs, W1).reshape(2 * N, _D // 2)
    a1 = _aggregate_full(h0, srclo3, srchi3, dst3)
    h1 = _update_matmul(a1, cd, cs, b1.reshape(1, -1),
                        W2).reshape(2 * N, _D // 2)
    a2 = _aggregate_full(h1, srclo3, srchi3, dst3)

    W3p = jnp.pad(W3, ((0, 0), (0, 8)))
    b3p = jnp.pad(b3, (0, 8))
    h2 = _update_matmul(a2, cd, cs, b2.reshape(1, -1), W3p)
    a3 = _aggregate(h2, src3, dst3)
    out = _finalize(a3, cd, b3p.reshape(1, -1))
    return out


# speedup vs baseline: 1.2202x; 1.0146x over previous
import functools

import jax
import jax.numpy as jnp
from jax import lax
from jax.experimental import pallas as pl
from jax.experimental.pallas import tpu as pltpu
from jax.experimental.pallas import tpu_sc as plsc

N = 10000
E = 320000
NC, NS, L = 2, 16, 16
NW = NC * NS
EPW = E // NW
K = 80
NCHUNK = EPW // K
ROWS_PER_SUB = N // NS
ZROUNDS = (N // K + NS - 1) // NS

_mesh = plsc.VectorSubcoreMesh(core_axis_name="c", subcore_axis_name="s")
_sc_params = pltpu.CompilerParams(use_tc_tiling_on_sc=False)


def _degrees(src3, dst3):
    out_type = (jax.ShapeDtypeStruct((NC, N, L), jnp.float32),
                jax.ShapeDtypeStruct((NC, N, L), jnp.float32))

    @functools.partial(
        pl.kernel, out_type=out_type, mesh=_mesh, compiler_params=_sc_params,
        scratch_types=[
            pltpu.VMEM((NCHUNK, K), jnp.int32),
            pltpu.VMEM((NCHUNK, K), jnp.int32),
            pltpu.VMEM((K, L), jnp.float32),
            pltpu.VMEM((K, L), jnp.float32),
            pltpu.VMEM_SHARED((N, L), jnp.float32),
            pltpu.VMEM_SHARED((N, L), jnp.float32),
            pltpu.SemaphoreType.DMA,
            pltpu.SemaphoreType.DMA,
            pltpu.SemaphoreType.DMA,
            pltpu.SemaphoreType.DMA,
            pltpu.SemaphoreType.DMA,
        ])
    def deg_kernel(src_hbm, dst_hbm, os_hbm, od_hbm,
                   srcv, dstv, onesv, zerov, accs, accd,
                   sem, ssem, dsem, ssem2, dsem2):
        ci = lax.axis_index("c")
        si = lax.axis_index("s")
        wid = si * NC + ci

        @pl.loop(0, K)
        def _(r):
            onesv[r, :] = jnp.ones((L,), jnp.float32)
            zerov[r, :] = jnp.zeros((L,), jnp.float32)

        @pl.loop(0, ZROUNDS)
        def _(k):
            chunk = si + NS * k

            @pl.when(chunk < N // K)
            def _():
                pltpu.async_copy(zerov, accs.at[pl.ds(chunk * K, K)], sem)
                pltpu.async_copy(zerov, accd.at[pl.ds(chunk * K, K)], sem)

        @pl.loop(0, ZROUNDS)
        def _(k):
            chunk = si + NS * k

            @pl.when(chunk < N // K)
            def _():
                pltpu.make_async_copy(zerov, accs.at[pl.ds(chunk * K, K)], sem).wait()
                pltpu.make_async_copy(zerov, accd.at[pl.ds(chunk * K, K)], sem).wait()

        plsc.subcore_barrier()
        pltpu.sync_copy(src_hbm.at[wid], srcv)
        pltpu.sync_copy(dst_hbm.at[wid], dstv)

        pltpu.async_copy(onesv, accs.at[srcv.at[0]], ssem, add=True)
        pltpu.async_copy(onesv, accd.at[dstv.at[0]], dsem, add=True)

        @pl.loop(0, NCHUNK, step=2)
        def _(c):
            @pl.when(c + 1 < NCHUNK)
            def _():
                pltpu.async_copy(onesv, accs.at[srcv.at[c + 1]], ssem2, add=True)
                pltpu.async_copy(onesv, accd.at[dstv.at[c + 1]], dsem2, add=True)

            pltpu.make_async_copy(onesv, accs.at[srcv.at[c]], ssem).wait()
            pltpu.make_async_copy(onesv, accd.at[dstv.at[c]], dsem).wait()

            @pl.when(c + 2 < NCHUNK)
            def _():
                pltpu.async_copy(onesv, accs.at[srcv.at[c + 2]], ssem, add=True)
                pltpu.async_copy(onesv, accd.at[dstv.at[c + 2]], dsem, add=True)

            @pl.when(c + 1 < NCHUNK)
            def _():
                pltpu.make_async_copy(onesv, accs.at[srcv.at[c + 1]], ssem2).wait()
                pltpu.make_async_copy(onesv, accd.at[dstv.at[c + 1]], dsem2).wait()

        plsc.subcore_barrier()

        @pl.loop(0, ZROUNDS)
        def _(k):
            chunk = si + NS * k

            @pl.when(chunk < N // K)
            def _():
                base = chunk * K
                pltpu.async_copy(accs.at[pl.ds(base, K)],
                                 os_hbm.at[ci, pl.ds(base, K)], sem)
                pltpu.async_copy(accd.at[pl.ds(base, K)],
                                 od_hbm.at[ci, pl.ds(base, K)], sem)

        @pl.loop(0, ZROUNDS)
        def _(k):
            chunk = si + NS * k

            @pl.when(chunk < N // K)
            def _():
                base = chunk * K
                pltpu.make_async_copy(accs.at[pl.ds(base, K)],
                                      os_hbm.at[ci, pl.ds(base, K)], sem).wait()
                pltpu.make_async_copy(accd.at[pl.ds(base, K)],
                                      od_hbm.at[ci, pl.ds(base, K)], sem).wait()

    return deg_kernel(src3, dst3)


_SEC = 25
_NSEC = NCHUNK // _SEC
_D = 128


def _aggregate_full(h2, srclo3, srchi3, dst3):
    H = _D // 2
    out_type = jax.ShapeDtypeStruct((NC, N, _D), jnp.float32)

    @functools.partial(
        pl.kernel, out_type=out_type, mesh=_mesh, compiler_params=_sc_params,
        scratch_types=[
            pltpu.VMEM((_SEC, K), jnp.int32),
            pltpu.VMEM((_SEC, K), jnp.int32),
            pltpu.VMEM((_SEC, K), jnp.int32),
            pltpu.VMEM((K, H), jnp.float32),
            pltpu.VMEM((K, H), jnp.float32),
            pltpu.VMEM((K, H), jnp.float32),
            pltpu.VMEM((K, H), jnp.float32),
            pltpu.VMEM((K, H), jnp.float32),
            pltpu.VMEM((K, H), jnp.float32),
            pltpu.VMEM_SHARED((N, H), jnp.float32),
            pltpu.VMEM_SHARED((N, H), jnp.float32),
            pltpu.SemaphoreType.DMA,
            pltpu.SemaphoreType.DMA,
            pltpu.SemaphoreType.DMA,
            pltpu.SemaphoreType.DMA,
            pltpu.SemaphoreType.DMA,
            pltpu.SemaphoreType.DMA,
            pltpu.SemaphoreType.DMA,
            pltpu.SemaphoreType.DMA,
            pltpu.SemaphoreType.DMA,
            pltpu.SemaphoreType.DMA,
            pltpu.SemaphoreType.DMA,
            pltpu.SemaphoreType.DMA,
        ])
    def agg2_kernel(h_hbm, srclo_hbm, srchi_hbm, dst_hbm, out_hbm,
                    srclov, srchiv, dstv, la, lb, lc, ha, hb, hc,
                    acc_lo, acc_hi,
                    sla, slb, slc, sha, shb, shc,
                    ala, alb, alc, aha, ahb, ahc):
        ci = lax.axis_index("c")
        si = lax.axis_index("s")
        wid = si * NC + ci

        @pl.loop(0, K)
        def _(r):
            @pl.loop(0, H, step=L)
            def _(j):
                la[r, pl.ds(j, L)] = jnp.zeros((L,), jnp.float32)

        @pl.loop(0, ZROUNDS)
        def _(k):
            chunk = si + NS * k

            @pl.when(chunk < N // K)
            def _():
                pltpu.async_copy(la, acc_lo.at[pl.ds(chunk * K, K)], ala)
                pltpu.async_copy(la, acc_hi.at[pl.ds(chunk * K, K)], aha)

        @pl.loop(0, ZROUNDS)
        def _(k):
            chunk = si + NS * k

            @pl.when(chunk < N // K)
            def _():
                pltpu.make_async_copy(la, acc_lo.at[pl.ds(chunk * K, K)], ala).wait()
                pltpu.make_async_copy(la, acc_hi.at[pl.ds(chunk * K, K)], aha).wait()

        plsc.subcore_barrier()

        @pl.loop(0, _NSEC)
        def _(s):
            pltpu.sync_copy(srclo_hbm.at[wid, pl.ds(s * _SEC, _SEC)], srclov)
            pltpu.sync_copy(srchi_hbm.at[wid, pl.ds(s * _SEC, _SEC)], srchiv)
            pltpu.sync_copy(dst_hbm.at[wid, pl.ds(s * _SEC, _SEC)], dstv)

            pltpu.async_copy(h_hbm.at[srclov.at[0]], la, sla)
            pltpu.async_copy(h_hbm.at[srchiv.at[0]], ha, sha)
            pltpu.async_copy(h_hbm.at[srclov.at[1]], lb, slb)
            pltpu.async_copy(h_hbm.at[srchiv.at[1]], hb, shb)
            pltpu.async_copy(h_hbm.at[srclov.at[2]], lc, slc)
            pltpu.async_copy(h_hbm.at[srchiv.at[2]], hc, shc)

            @pl.loop(0, _SEC, step=3)
            def _(c):
                pltpu.make_async_copy(h_hbm.at[srclov.at[c]], la, sla).wait()
                pltpu.async_copy(la, acc_lo.at[dstv.at[c]], ala, add=True)
                pltpu.make_async_copy(h_hbm.at[srchiv.at[c]], ha, sha).wait()
                pltpu.async_copy(ha, acc_hi.at[dstv.at[c]], aha, add=True)

                @pl.when(c + 1 < _SEC)
                def _():
                    pltpu.make_async_copy(h_hbm.at[srclov.at[c + 1]], lb, slb).wait()
                    pltpu.async_copy(lb, acc_lo.at[dstv.at[c + 1]], alb, add=True)
                    pltpu.make_async_copy(h_hbm.at[srchiv.at[c + 1]], hb, shb).wait()
                    pltpu.async_copy(hb, acc_hi.at[dstv.at[c + 1]], ahb, add=True)

                pltpu.make_async_copy(la, acc_lo.at[dstv.at[c]], ala).wait()
                pltpu.make_async_copy(ha, acc_hi.at[dstv.at[c]], aha).wait()

                @pl.when(c + 3 < _SEC)
                def _():
                    pltpu.async_copy(h_hbm.at[srclov.at[c + 3]], la, sla)
                    pltpu.async_copy(h_hbm.at[srchiv.at[c + 3]], ha, sha)

                @pl.when(c + 2 < _SEC)
                def _():
                    pltpu.make_async_copy(h_hbm.at[srclov.at[c + 2]], lc, slc).wait()
                    pltpu.async_copy(lc, acc_lo.at[dstv.at[c + 2]], alc, add=True)
                    pltpu.make_async_copy(h_hbm.at[srchiv.at[c + 2]], hc, shc).wait()
                    pltpu.async_copy(hc, acc_hi.at[dstv.at[c + 2]], ahc, add=True)

                @pl.when(c + 1 < _SEC)
                def _():
                    pltpu.make_async_copy(lb, acc_lo.at[dstv.at[c + 1]], alb).wait()
                    pltpu.make_async_copy(hb, acc_hi.at[dstv.at[c + 1]], ahb).wait()

                @pl.when(c + 4 < _SEC)
                def _():
                    pltpu.async_copy(h_hbm.at[srclov.at[c + 4]], lb, slb)
                    pltpu.async_copy(h_hbm.at[srchiv.at[c + 4]], hb, shb)

                @pl.when(c + 2 < _SEC)
                def _():
                    pltpu.make_async_copy(lc, acc_lo.at[dstv.at[c + 2]], alc).wait()
                    pltpu.make_async_copy(hc, acc_hi.at[dstv.at[c + 2]], ahc).wait()

                @pl.when(c + 5 < _SEC)
                def _():
                    pltpu.async_copy(h_hbm.at[srclov.at[c + 5]], lc, slc)
                    pltpu.async_copy(h_hbm.at[srchiv.at[c + 5]], hc, shc)

        plsc.subcore_barrier()

        @pl.loop(0, ZROUNDS)
        def _(k):
            chunk = si + NS * k

            @pl.when(chunk < N // K)
            def _():
                base = chunk * K
                pltpu.async_copy(acc_lo.at[pl.ds(base, K)],
                                 out_hbm.at[ci, pl.ds(base, K), pl.ds(0, H)], ala)
                pltpu.async_copy(acc_hi.at[pl.ds(base, K)],
                                 out_hbm.at[ci, pl.ds(base, K), pl.ds(H, H)], aha)

        @pl.loop(0, ZROUNDS)
        def _(k):
            chunk = si + NS * k

            @pl.when(chunk < N // K)
            def _():
                base = chunk * K
                pltpu.make_async_copy(
                    acc_lo.at[pl.ds(base, K)],
                    out_hbm.at[ci, pl.ds(base, K), pl.ds(0, H)], ala).wait()
                pltpu.make_async_copy(
                    acc_hi.at[pl.ds(base, K)],
                    out_hbm.at[ci, pl.ds(base, K), pl.ds(H, H)], aha).wait()

    return agg2_kernel(h2, srclo3, srchi3, dst3)


def _aggregate(h, src3, dst3):
    D = h.shape[1]

    @functools.partial(
        pl.kernel, out_type=jax.ShapeDtypeStruct((NC, N, D), jnp.float32),
        mesh=_mesh, compiler_params=_sc_params,
        scratch_types=[
            pltpu.VMEM((NCHUNK, K), jnp.int32),
            pltpu.VMEM((NCHUNK, K), jnp.int32),
            pltpu.VMEM((K, D), jnp.float32),
            pltpu.VMEM((K, D), jnp.float32),
            pltpu.VMEM((K, D), jnp.float32),
            pltpu.VMEM_SHARED((N, D), jnp.float32),
            pltpu.SemaphoreType.DMA,
            pltpu.SemaphoreType.DMA,
            pltpu.SemaphoreType.DMA,
            pltpu.SemaphoreType.DMA,
            pltpu.SemaphoreType.DMA,
            pltpu.SemaphoreType.DMA,
        ])
    def agg_kernel(h_hbm, src_hbm, dst_hbm, out_hbm,
                   srcv, dstv, bufa, bufb, bufc, accum,
                   sema, semb, semc, aa, ab, ac):
        ci = lax.axis_index("c")
        si = lax.axis_index("s")
        wid = si * NC + ci

        @pl.loop(0, K)
        def _(r):
            @pl.loop(0, D, step=L)
            def _(j):
                bufa[r, pl.ds(j, L)] = jnp.zeros((L,), jnp.float32)

        @pl.loop(0, ZROUNDS)
        def _(k):
            chunk = si + NS * k

            @pl.when(chunk < N // K)
            def _():
                pltpu.async_copy(bufa, accum.at[pl.ds(chunk * K, K)], aa)

        @pl.loop(0, ZROUNDS)
        def _(k):
            chunk = si + NS * k

            @pl.when(chunk < N // K)
            def _():
                pltpu.make_async_copy(bufa, accum.at[pl.ds(chunk * K, K)], aa).wait()

        plsc.subcore_barrier()
        pltpu.sync_copy(src_hbm.at[wid], srcv)
        pltpu.sync_copy(dst_hbm.at[wid], dstv)

        pltpu.async_copy(h_hbm.at[srcv.at[0]], bufa, sema)
        pltpu.async_copy(h_hbm.at[srcv.at[1]], bufb, semb)
        pltpu.async_copy(h_hbm.at[srcv.at[2]], bufc, semc)

        @pl.loop(0, NCHUNK, step=3)
        def _(c):
            pltpu.make_async_copy(h_hbm.at[srcv.at[c]], bufa, sema).wait()
            pltpu.async_copy(bufa, accum.at[dstv.at[c]], aa, add=True)

            @pl.when(c + 1 < NCHUNK)
            def _():
                pltpu.make_async_copy(h_hbm.at[srcv.at[c + 1]], bufb, semb).wait()
                pltpu.async_copy(bufb, accum.at[dstv.at[c + 1]], ab, add=True)

            pltpu.make_async_copy(bufa, accum.at[dstv.at[c]], aa).wait()

            @pl.when(c + 3 < NCHUNK)
            def _():
                pltpu.async_copy(h_hbm.at[srcv.at[c + 3]], bufa, sema)

            @pl.when(c + 2 < NCHUNK)
            def _():
                pltpu.make_async_copy(h_hbm.at[srcv.at[c + 2]], bufc, semc).wait()
                pltpu.async_copy(bufc, accum.at[dstv.at[c + 2]], ac, add=True)

            @pl.when(c + 1 < NCHUNK)
            def _():
                pltpu.make_async_copy(bufb, accum.at[dstv.at[c + 1]], ab).wait()

            @pl.when(c + 4 < NCHUNK)
            def _():
                pltpu.async_copy(h_hbm.at[srcv.at[c + 4]], bufb, semb)

            @pl.when(c + 2 < NCHUNK)
            def _():
                pltpu.make_async_copy(bufc, accum.at[dstv.at[c + 2]], ac).wait()

            @pl.when(c + 5 < NCHUNK)
            def _():
                pltpu.async_copy(h_hbm.at[srcv.at[c + 5]], bufc, semc)

        plsc.subcore_barrier()

        @pl.loop(0, ZROUNDS)
        def _(k):
            chunk = si + NS * k

            @pl.when(chunk < N // K)
            def _():
                base = chunk * K
                pltpu.async_copy(accum.at[pl.ds(base, K)],
                                 out_hbm.at[ci, pl.ds(base, K)], aa)

        @pl.loop(0, ZROUNDS)
        def _(k):
            chunk = si + NS * k

            @pl.when(chunk < N // K)
            def _():
                base = chunk * K
                pltpu.make_async_copy(accum.at[pl.ds(base, K)],
                                      out_hbm.at[ci, pl.ds(base, K)], aa).wait()

    return agg_kernel(h, src3, dst3)


_R = 1000


def _norm_from(counts_ref):
    c = counts_ref[0, :, 0:1] + counts_ref[1, :, 0:1]
    return lax.rsqrt(jnp.maximum(c, 1.0))


def _scale_matmul(x, cs, W):
    D, Do = W.shape

    def body(x_ref, cs_ref, w_ref, o_ref):
        o_ref[...] = jnp.dot(x_ref[...] * _norm_from(cs_ref), w_ref[...],
                             preferred_element_type=jnp.float32)

    return pl.pallas_call(
        body, grid=(N // _R,),
        in_specs=[pl.BlockSpec((_R, D), lambda i: (i, 0)),
                  pl.BlockSpec((NC, _R, L), lambda i: (0, i, 0)),
                  pl.BlockSpec((D, Do), lambda i: (0, 0))],
        out_specs=pl.BlockSpec((_R, Do), lambda i: (i, 0)),
        out_shape=jax.ShapeDtypeStruct((N, Do), jnp.float32))(x, cs, W)


def _update_matmul(agg, cd, cs, b, W):
    D, Do = W.shape

    def body(a_ref, cd_ref, cs_ref, b_ref, w_ref, o_ref):
        a = a_ref[0] + a_ref[1]
        h = jnp.maximum(a * _norm_from(cd_ref) + b_ref[...], 0.0) \
            * _norm_from(cs_ref)
        o_ref[...] = jnp.dot(h, w_ref[...], preferred_element_type=jnp.float32)

    return pl.pallas_call(
        body, grid=(N // _R,),
        in_specs=[pl.BlockSpec((NC, _R, D), lambda i: (0, i, 0)),
                  pl.BlockSpec((NC, _R, L), lambda i: (0, i, 0)),
                  pl.BlockSpec((NC, _R, L), lambda i: (0, i, 0)),
                  pl.BlockSpec((1, D), lambda i: (0, 0)),
                  pl.BlockSpec((D, Do), lambda i: (0, 0))],
        out_specs=pl.BlockSpec((_R, Do), lambda i: (i, 0)),
        out_shape=jax.ShapeDtypeStruct((N, Do), jnp.float32))(agg, cd, cs, b, W)


def _finalize(agg, cd, b):
    D = agg.shape[2]
    Do = 40

    def body(a_ref, cd_ref, b_ref, o_ref):
        a = a_ref[0] + a_ref[1]
        o_ref[...] = (a * _norm_from(cd_ref) + b_ref[...])[:, :Do]

    return pl.pallas_call(
        body, grid=(N // _R,),
        in_specs=[pl.BlockSpec((NC, _R, D), lambda i: (0, i, 0)),
                  pl.BlockSpec((NC, _R, L), lambda i: (0, i, 0)),
                  pl.BlockSpec((1, D), lambda i: (0, 0))],
        out_specs=pl.BlockSpec((_R, Do), lambda i: (i, 0)),
        out_shape=jax.ShapeDtypeStruct((N, Do), jnp.float32))(agg, cd, b)


def kernel(features, edge_index, W1, b1, W2, b2, W3, b3):
    src = edge_index[0].astype(jnp.int32)
    src3 = src.reshape(NW, NCHUNK, K)
    dst3 = edge_index[1].astype(jnp.int32).reshape(NW, NCHUNK, K)
    srclo3 = (src * 2).reshape(NW, NCHUNK, K)
    srchi3 = srclo3 + 1

    cs, cd = _degrees(src3, dst3)

    h0 = _scale_matmul(features, cs, W1).reshape(2 * N, _D // 2)
    a1 = _aggregate_full(h0, srclo3, srchi3, dst3)
    h1 = _update_matmul(a1, cd, cs, b1.reshape(1, -1),
                        W2).reshape(2 * N, _D // 2)
    a2 = _aggregate_full(h1, srclo3, srchi3, dst3)

    W3p = jnp.pad(W3, ((0, 0), (0, 8)))
    b3p = jnp.pad(b3, (0, 8))
    h2 = _update_matmul(a2, cd, cs, b2.reshape(1, -1), W3p)
    a3 = _aggregate(h2, src3, dst3)
    out = _finalize(a3, cd, b3p.reshape(1, -1))
    return out
